# Initial kernel scaffold; baseline (speedup 1.0000x reference)
import jax
import jax.numpy as jnp
from jax.experimental import pallas as pl


def kernel(x, edge_index, op1, op2, W1a, b1a, W1b, b1b, W2a, b2a, W2b, b2b):
    raise NotImplementedError("write your pallas kernel here")


# speedup vs baseline: 8.7037x; 8.7037x over previous
import functools

import jax
import jax.numpy as jnp
from jax import lax
from jax.experimental import pallas as pl
from jax.experimental.pallas import tpu as pltpu
from jax.experimental.pallas import tpu_sc as plsc

N = 10000
E = 320000
D = 128

NC = 2
NS = 16
NW = NC * NS
PER_TILE = E // NW
CH = 80
N_CHUNK = PER_TILE // CH
NP = 10112
ROWS_PER_TILE = NP // NS

_MESH = plsc.VectorSubcoreMesh(core_axis_name="c", subcore_axis_name="s")

DEG_W = 128


@functools.partial(
    pl.kernel,
    out_type=jax.ShapeDtypeStruct((NC, NP, DEG_W), jnp.float32),
    mesh=_MESH,
    scratch_types=[
        pltpu.VMEM((CH,), jnp.int32),
        pltpu.VMEM((CH, DEG_W), jnp.float32),
        pltpu.VMEM_SHARED((NP, DEG_W), jnp.float32),
    ],
)
def _sc_degree(dst_hbm, ones_hbm, zeros_hbm, out_hbm, idx_v, ones_v, acc_s):
    c = lax.axis_index("c")
    s = lax.axis_index("s")
    wid = s * NC + c
    pltpu.sync_copy(zeros_hbm, acc_s.at[pl.ds(s * ROWS_PER_TILE, ROWS_PER_TILE)])
    pltpu.sync_copy(ones_hbm, ones_v)
    plsc.subcore_barrier()
    base = wid * PER_TILE

    def body(i, carry):
        off = base + i * CH
        pltpu.sync_copy(dst_hbm.at[pl.ds(off, CH)], idx_v)
        pltpu.sync_copy(ones_v, acc_s.at[idx_v], add=True)
        return carry

    lax.fori_loop(0, N_CHUNK, body, 0)
    plsc.subcore_barrier()
    pltpu.sync_copy(
        acc_s.at[pl.ds(s * ROWS_PER_TILE, ROWS_PER_TILE)],
        out_hbm.at[c, pl.ds(s * ROWS_PER_TILE, ROWS_PER_TILE)],
    )


@functools.partial(
    pl.kernel,
    out_type=jax.ShapeDtypeStruct((NC, NP, D), jnp.float32),
    mesh=_MESH,
    scratch_types=[
        pltpu.VMEM((CH,), jnp.int32),
        pltpu.VMEM((CH,), jnp.int32),
        pltpu.VMEM((CH, D), jnp.float32),
        pltpu.VMEM_SHARED((NP, D), jnp.float32),
        pltpu.SemaphoreType.DMA,
    ],
)
def _sc_edge_pass(z_hbm, src_hbm, dst_hbm, zeros_hbm, out_hbm,
                  src_v, dst_v, rows_v, acc_s, sem):
    c = lax.axis_index("c")
    s = lax.axis_index("s")
    wid = s * NC + c
    pltpu.sync_copy(zeros_hbm, acc_s.at[pl.ds(s * ROWS_PER_TILE, ROWS_PER_TILE)])
    plsc.subcore_barrier()
    base = wid * PER_TILE

    def body(i, carry):
        off = base + i * CH
        pltpu.sync_copy(src_hbm.at[pl.ds(off, CH)], src_v)
        pltpu.sync_copy(dst_hbm.at[pl.ds(off, CH)], dst_v)
        pltpu.async_copy(z_hbm.at[src_v], rows_v, sem).wait()
        pltpu.sync_copy(rows_v, acc_s.at[dst_v], add=True)
        return carry

    lax.fori_loop(0, N_CHUNK, body, 0)
    plsc.subcore_barrier()
    pltpu.sync_copy(
        acc_s.at[pl.ds(s * ROWS_PER_TILE, ROWS_PER_TILE)],
        out_hbm.at[c, pl.ds(s * ROWS_PER_TILE, ROWS_PER_TILE)],
    )


ROW_BLK = 400
N_BLK = N // ROW_BLK


def _tc_dinv(deg_part):
    def body(p_ref, o_ref):
        deg = p_ref[0, :, 0:1] + p_ref[1, :, 0:1] + 1.0
        o_ref[...] = lax.rsqrt(deg)

    return pl.pallas_call(
        body,
        grid=(N_BLK,),
        in_specs=[pl.BlockSpec((NC, ROW_BLK, DEG_W), lambda i: (0, i, 0))],
        out_specs=pl.BlockSpec((ROW_BLK, 1), lambda i: (i, 0)),
        out_shape=jax.ShapeDtypeStruct((N, 1), jnp.float32),
    )(deg_part)


def _tc_pre(x, dinv, W):
    def body(x_ref, d_ref, w_ref, o_ref):
        o_ref[...] = jnp.dot(d_ref[...] * x_ref[...], w_ref[...],
                             preferred_element_type=jnp.float32)

    return pl.pallas_call(
        body,
        grid=(N_BLK,),
        in_specs=[
            pl.BlockSpec((ROW_BLK, D), lambda i: (i, 0)),
            pl.BlockSpec((ROW_BLK, 1), lambda i: (i, 0)),
            pl.BlockSpec((D, D), lambda i: (0, 0)),
        ],
        out_specs=pl.BlockSpec((ROW_BLK, D), lambda i: (i, 0)),
        out_shape=jax.ShapeDtypeStruct((N, D), jnp.float32),
    )(x, dinv, W)


def _tc_mid(part, z, dinv, b, W):
    def body(p_ref, z_ref, d_ref, b_ref, w_ref, o_ref):
        agg = p_ref[0] + p_ref[1] + z_ref[...]
        h = jnp.maximum(d_ref[...] * agg + b_ref[...], 0.0)
        o_ref[...] = d_ref[...] * jnp.dot(h, w_ref[...],
                                          preferred_element_type=jnp.float32)

    return pl.pallas_call(
        body,
        grid=(N_BLK,),
        in_specs=[
            pl.BlockSpec((NC, ROW_BLK, D), lambda i: (0, i, 0)),
            pl.BlockSpec((ROW_BLK, D), lambda i: (i, 0)),
            pl.BlockSpec((ROW_BLK, 1), lambda i: (i, 0)),
            pl.BlockSpec((1, D), lambda i: (0, 0)),
            pl.BlockSpec((D, D), lambda i: (0, 0)),
        ],
        out_specs=pl.BlockSpec((ROW_BLK, D), lambda i: (i, 0)),
        out_shape=jax.ShapeDtypeStruct((N, D), jnp.float32),
    )(part, z, dinv, b, W)


def _tc_post(part, z, dinv, b):
    def body(p_ref, z_ref, d_ref, b_ref, o_ref):
        agg = p_ref[0] + p_ref[1] + z_ref[...]
        o_ref[...] = d_ref[...] * agg + b_ref[...]

    return pl.pallas_call(
        body,
        grid=(N_BLK,),
        in_specs=[
            pl.BlockSpec((NC, ROW_BLK, D), lambda i: (0, i, 0)),
            pl.BlockSpec((ROW_BLK, D), lambda i: (i, 0)),
            pl.BlockSpec((ROW_BLK, 1), lambda i: (i, 0)),
            pl.BlockSpec((1, D), lambda i: (0, 0)),
        ],
        out_specs=pl.BlockSpec((ROW_BLK, D), lambda i: (i, 0)),
        out_shape=jax.ShapeDtypeStruct((N, D), jnp.float32),
    )(part, z, dinv, b)


def _tc_final_dot(y1, y2, ops):
    def body(ops_ref, y1_ref, y2_ref, o_ref):
        o1 = ops_ref[0]
        o2 = ops_ref[1]
        r1a = y1_ref[pl.ds(o1, 1), :]
        r1b = y1_ref[pl.ds(o2, 1), :]
        r2a = y2_ref[pl.ds(o1, 1), :]
        r2b = y2_ref[pl.ds(o2, 1), :]
        o_ref[0] = jnp.sum(r1a * r1b)
        o_ref[1] = jnp.sum(r2a * r2b)

    return pl.pallas_call(
        body,
        in_specs=[
            pl.BlockSpec(memory_space=pltpu.SMEM),
            pl.BlockSpec(memory_space=pltpu.VMEM),
            pl.BlockSpec(memory_space=pltpu.VMEM),
        ],
        out_specs=pl.BlockSpec(memory_space=pltpu.SMEM),
        out_shape=jax.ShapeDtypeStruct((2,), jnp.float32),
    )(ops, y1, y2)


def kernel(x, edge_index, op1, op2, W1a, b1a, W1b, b1b, W2a, b2a, W2b, b2b):
    src = edge_index[0]
    dst = edge_index[1]
    ones_deg = jnp.ones((CH, DEG_W), jnp.float32)
    zeros_deg = jnp.zeros((ROWS_PER_TILE, DEG_W), jnp.float32)
    zeros_row = jnp.zeros((ROWS_PER_TILE, D), jnp.float32)

    deg_part = _sc_degree(dst, ones_deg, zeros_deg)
    dinv = _tc_dinv(deg_part)

    def stack(Wa, ba, Wb, bb):
        z1 = _tc_pre(x, dinv, Wa)
        p1 = _sc_edge_pass(z1, src, dst, zeros_row)
        z2 = _tc_mid(p1, z1, dinv, ba.reshape(1, D), Wb)
        p2 = _sc_edge_pass(z2, src, dst, zeros_row)
        return _tc_post(p2, z2, dinv, bb.reshape(1, D))

    y1 = stack(W1a, b1a, W1b, b1b)
    y2 = stack(W2a, b2a, W2b, b2b)
    ops = jnp.stack([op1, op2]).astype(jnp.int32)
    return _tc_final_dot(y1, y2, ops)


# speedup vs baseline: 13.7896x; 1.5843x over previous
import functools

import jax
import jax.numpy as jnp
from jax import lax
from jax.experimental import pallas as pl
from jax.experimental.pallas import tpu as pltpu
from jax.experimental.pallas import tpu_sc as plsc

N = 10000
E = 320000
D = 128

NC = 2
NS = 16
NW = NC * NS
PER_TILE = E // NW
CH = 80
N_CHUNK = PER_TILE // CH
NP = 10112
ROWS_PER_TILE = NP // NS

_MESH = plsc.VectorSubcoreMesh(core_axis_name="c", subcore_axis_name="s")

DEG_W = 128


@functools.partial(
    pl.kernel,
    out_type=jax.ShapeDtypeStruct((NC, NP, DEG_W), jnp.float32),
    mesh=_MESH,
    scratch_types=[
        pltpu.VMEM((CH,), jnp.int32),
        pltpu.VMEM((CH, DEG_W), jnp.float32),
        pltpu.VMEM_SHARED((NP, DEG_W), jnp.float32),
    ],
)
def _sc_degree(dst_hbm, ones_hbm, zeros_hbm, out_hbm, idx_v, ones_v, acc_s):
    c = lax.axis_index("c")
    s = lax.axis_index("s")
    wid = s * NC + c
    pltpu.sync_copy(zeros_hbm, acc_s.at[pl.ds(s * ROWS_PER_TILE, ROWS_PER_TILE)])
    pltpu.sync_copy(ones_hbm, ones_v)
    plsc.subcore_barrier()
    base = wid * PER_TILE

    def body(i, carry):
        off = base + i * CH
        pltpu.sync_copy(dst_hbm.at[pl.ds(off, CH)], idx_v)
        pltpu.sync_copy(ones_v, acc_s.at[idx_v], add=True)
        return carry

    lax.fori_loop(0, N_CHUNK, body, 0)
    plsc.subcore_barrier()
    pltpu.sync_copy(
        acc_s.at[pl.ds(s * ROWS_PER_TILE, ROWS_PER_TILE)],
        out_hbm.at[c, pl.ds(s * ROWS_PER_TILE, ROWS_PER_TILE)],
    )


@functools.partial(
    pl.kernel,
    out_type=jax.ShapeDtypeStruct((NC, NP, D), jnp.float32),
    mesh=_MESH,
    scratch_types=[
        pltpu.VMEM((CH,), jnp.int32),
        pltpu.VMEM((CH,), jnp.int32),
        pltpu.VMEM((CH, D), jnp.float32),
        pltpu.VMEM_SHARED((NP, D), jnp.float32),
        pltpu.SemaphoreType.DMA,
    ],
)
def _sc_edge_pass(z_hbm, src_hbm, dst_hbm, zeros_hbm, out_hbm,
                  src_v, dst_v, rows_v, acc_s, sem):
    c = lax.axis_index("c")
    s = lax.axis_index("s")
    wid = s * NC + c
    pltpu.sync_copy(zeros_hbm, acc_s.at[pl.ds(s * ROWS_PER_TILE, ROWS_PER_TILE)])
    plsc.subcore_barrier()
    base = wid * PER_TILE

    def body(i, carry):
        off = base + i * CH
        pltpu.sync_copy(src_hbm.at[pl.ds(off, CH)], src_v)
        pltpu.sync_copy(dst_hbm.at[pl.ds(off, CH)], dst_v)
        pltpu.async_copy(z_hbm.at[src_v], rows_v, sem).wait()
        pltpu.sync_copy(rows_v, acc_s.at[dst_v], add=True)
        return carry

    lax.fori_loop(0, N_CHUNK, body, 0)
    plsc.subcore_barrier()
    pltpu.sync_copy(
        acc_s.at[pl.ds(s * ROWS_PER_TILE, ROWS_PER_TILE)],
        out_hbm.at[c, pl.ds(s * ROWS_PER_TILE, ROWS_PER_TILE)],
    )


L = 16
NVEC = PER_TILE // L
LCAP = PER_TILE + 2 * L
TRASH = LCAP - 1


@functools.partial(
    pl.kernel,
    out_type=jax.ShapeDtypeStruct((NC, NS, 8, D), jnp.float32),
    mesh=_MESH,
    compiler_params=pltpu.CompilerParams(needs_layout_passes=False),
    scratch_types=[
        pltpu.VMEM((PER_TILE,), jnp.int32),
        pltpu.VMEM((PER_TILE,), jnp.int32),
        pltpu.VMEM((LCAP,), jnp.int32),
        pltpu.VMEM((LCAP,), jnp.int32),
        pltpu.VMEM((2, L), jnp.int32),
        pltpu.VMEM((L,), jnp.int32),
        pltpu.VMEM((L, D), jnp.float32),
        pltpu.VMEM((L, D), jnp.float32),
        pltpu.VMEM((8, D), jnp.float32),
        pltpu.SemaphoreType.DMA,
    ],
)
def _sc_filter(src_hbm, dst_hbm, ops_hbm, q1_hbm, q2_hbm, out_hbm,
               dst_b, src_b, list1, list2, ops_v, pad_v, rows1, rows2, uacc, sem):
    c = lax.axis_index("c")
    s = lax.axis_index("s")
    wid = s * NC + c
    base = wid * PER_TILE
    pltpu.sync_copy(dst_hbm.at[pl.ds(base, PER_TILE)], dst_b)
    pltpu.sync_copy(src_hbm.at[pl.ds(base, PER_TILE)], src_b)
    pltpu.sync_copy(ops_hbm, ops_v)
    op1v = ops_v[0, :]
    op2v = ops_v[1, :]
    z16 = jnp.zeros((L,), jnp.float32)
    for r in range(8):
        for j in range(D // L):
            uacc[r, pl.ds(j * L, L)] = z16

    def scan_body(i, carry):
        cnt1, cnt2 = carry
        dv = dst_b[pl.ds(i * L, L)]
        sv = src_b[pl.ds(i * L, L)]
        m1 = dv == op1v
        m2 = dv == op2v
        one16 = jnp.ones((L,), jnp.int32)
        trash16 = jnp.full((L,), TRASH, jnp.int32)
        pos1 = plsc.cumsum(m1.astype(jnp.int32))
        pos2 = plsc.cumsum(m2.astype(jnp.int32))
        c1v = jnp.full((L,), cnt1, jnp.int32)
        c2v = jnp.full((L,), cnt2, jnp.int32)
        idx1 = jnp.where(m1, c1v + pos1 - one16, trash16)
        idx2 = jnp.where(m2, c2v + pos2 - one16, trash16)
        plsc.store_scatter(list1, [idx1], sv)
        plsc.store_scatter(list2, [idx2], sv)
        return cnt1 + jnp.max(pos1), cnt2 + jnp.max(pos2)

    cnt1, cnt2 = lax.fori_loop(0, NVEC, scan_body, (jnp.int32(0), jnp.int32(0)))

    zi16 = jnp.zeros((L,), jnp.int32)
    pad_v[...] = zi16

    def accumulate(lst, cnt, r1, r2):
        lst[pl.ds(cnt, L)] = zi16
        nch = (cnt + (L - 1)) // L

        def body(k, carry):
            ids = lst.at[pl.ds(k * L, L)]
            pltpu.async_copy(q1_hbm.at[ids], rows1, sem).wait()
            pltpu.async_copy(q2_hbm.at[ids], rows2, sem).wait()
            for j in range(D // L):
                sl = pl.ds(j * L, L)
                a1 = uacc[r1, sl]
                a2 = uacc[r2, sl]
                for t in range(L):
                    a1 = a1 + rows1[t, sl]
                    a2 = a2 + rows2[t, sl]
                uacc[r1, sl] = a1
                uacc[r2, sl] = a2
            return carry

        lax.fori_loop(0, nch, body, 0)
        nspv = jnp.full((L,), (nch * L - cnt).astype(jnp.float32))
        pltpu.async_copy(q1_hbm.at[pad_v], rows1, sem).wait()
        pltpu.async_copy(q2_hbm.at[pad_v], rows2, sem).wait()
        for j in range(D // L):
            sl = pl.ds(j * L, L)
            uacc[r1, sl] = uacc[r1, sl] - nspv * rows1[0, sl]
            uacc[r2, sl] = uacc[r2, sl] - nspv * rows2[0, sl]

    accumulate(list1, cnt1, 0, 2)
    accumulate(list2, cnt2, 1, 3)
    pltpu.sync_copy(uacc, out_hbm.at[c, s])


ROW_BLK = 400
N_BLK = N // ROW_BLK


def _tc_dinv(deg_part):
    def body(p_ref, o_ref):
        deg = p_ref[0, :, 0:1] + p_ref[1, :, 0:1] + 1.0
        o_ref[...] = lax.rsqrt(deg)

    return pl.pallas_call(
        body,
        grid=(N_BLK,),
        in_specs=[pl.BlockSpec((NC, ROW_BLK, DEG_W), lambda i: (0, i, 0))],
        out_specs=pl.BlockSpec((ROW_BLK, 1), lambda i: (i, 0)),
        out_shape=jax.ShapeDtypeStruct((N, 1), jnp.float32),
    )(deg_part)


def _tc_pre(x, dinv, W):
    def body(x_ref, d_ref, w_ref, o_ref):
        o_ref[...] = jnp.dot(d_ref[...] * x_ref[...], w_ref[...],
                             preferred_element_type=jnp.float32)

    return pl.pallas_call(
        body,
        grid=(N_BLK,),
        in_specs=[
            pl.BlockSpec((ROW_BLK, D), lambda i: (i, 0)),
            pl.BlockSpec((ROW_BLK, 1), lambda i: (i, 0)),
            pl.BlockSpec((D, D), lambda i: (0, 0)),
        ],
        out_specs=pl.BlockSpec((ROW_BLK, D), lambda i: (i, 0)),
        out_shape=jax.ShapeDtypeStruct((N, D), jnp.float32),
    )(x, dinv, W)


def _tc_q(part, z, dinv, b):
    def body(p_ref, z_ref, d_ref, b_ref, o_ref):
        agg = p_ref[0] + p_ref[1] + z_ref[...]
        h = jnp.maximum(d_ref[...] * agg + b_ref[...], 0.0)
        o_ref[...] = d_ref[...] * h

    return pl.pallas_call(
        body,
        grid=(N_BLK,),
        in_specs=[
            pl.BlockSpec((NC, ROW_BLK, D), lambda i: (0, i, 0)),
            pl.BlockSpec((ROW_BLK, D), lambda i: (i, 0)),
            pl.BlockSpec((ROW_BLK, 1), lambda i: (i, 0)),
            pl.BlockSpec((1, D), lambda i: (0, 0)),
        ],
        out_specs=pl.BlockSpec((ROW_BLK, D), lambda i: (i, 0)),
        out_shape=jax.ShapeDtypeStruct((N, D), jnp.float32),
    )(part, z, dinv, b)


def _tc_finish(slabs, q1, q2, dinv, W1b, b1b, W2b, b2b, ops):
    def body(ops_ref, sl_ref, q1_ref, q2_ref, d_ref, w1_ref, b1_ref,
             w2_ref, b2_ref, o_ref):
        u = jnp.sum(sl_ref[...], axis=0)
        o1 = ops_ref[0]
        o2 = ops_ref[1]
        u11 = u[0:1] + q1_ref[pl.ds(o1, 1), :]
        u12 = u[1:2] + q1_ref[pl.ds(o2, 1), :]
        u21 = u[2:3] + q2_ref[pl.ds(o1, 1), :]
        u22 = u[3:4] + q2_ref[pl.ds(o2, 1), :]
        d1 = d_ref[pl.ds(o1, 1), :]
        d2 = d_ref[pl.ds(o2, 1), :]
        a = jnp.dot(jnp.concatenate([u11, u12], axis=0), w1_ref[...],
                    preferred_element_type=jnp.float32)
        b = jnp.dot(jnp.concatenate([u21, u22], axis=0), w2_ref[...],
                    preferred_element_type=jnp.float32)
        y11 = d1 * a[0:1] + b1_ref[...]
        y12 = d2 * a[1:2] + b1_ref[...]
        y21 = d1 * b[0:1] + b2_ref[...]
        y22 = d2 * b[1:2] + b2_ref[...]
        o_ref[0] = jnp.sum(y11 * y12)
        o_ref[1] = jnp.sum(y21 * y22)

    return pl.pallas_call(
        body,
        in_specs=[
            pl.BlockSpec(memory_space=pltpu.SMEM),
            pl.BlockSpec(memory_space=pltpu.VMEM),
            pl.BlockSpec(memory_space=pltpu.VMEM),
            pl.BlockSpec(memory_space=pltpu.VMEM),
            pl.BlockSpec(memory_space=pltpu.VMEM),
            pl.BlockSpec(memory_space=pltpu.VMEM),
            pl.BlockSpec(memory_space=pltpu.VMEM),
            pl.BlockSpec(memory_space=pltpu.VMEM),
            pl.BlockSpec(memory_space=pltpu.VMEM),
        ],
        out_specs=pl.BlockSpec(memory_space=pltpu.SMEM),
        out_shape=jax.ShapeDtypeStruct((2,), jnp.float32),
    )(ops, slabs, q1, q2, dinv, W1b, b1b, W2b, b2b)


def kernel(x, edge_index, op1, op2, W1a, b1a, W1b, b1b, W2a, b2a, W2b, b2b):
    src = edge_index[0]
    dst = edge_index[1]
    ones_deg = jnp.ones((CH, DEG_W), jnp.float32)
    zeros_deg = jnp.zeros((ROWS_PER_TILE, DEG_W), jnp.float32)
    zeros_row = jnp.zeros((ROWS_PER_TILE, D), jnp.float32)
    ops = jnp.stack([op1, op2]).astype(jnp.int32)
    ops16 = jnp.broadcast_to(ops[:, None], (2, L))

    deg_part = _sc_degree(dst, ones_deg, zeros_deg)
    dinv = _tc_dinv(deg_part)

    z1 = _tc_pre(x, dinv, W1a)
    z2 = _tc_pre(x, dinv, W2a)
    p1 = _sc_edge_pass(z1, src, dst, zeros_row)
    p2 = _sc_edge_pass(z2, src, dst, zeros_row)
    q1 = _tc_q(p1, z1, dinv, b1a.reshape(1, D))
    q2 = _tc_q(p2, z2, dinv, b2a.reshape(1, D))

    slabs = _sc_filter(src, dst, ops16, q1, q2)
    slabs = slabs.reshape(NC * NS, 8, D)
    return _tc_finish(slabs, q1, q2, dinv, W1b, b1b.reshape(1, D),
                      W2b, b2b.reshape(1, D), ops)


# speedup vs baseline: 18.8716x; 1.3685x over previous
import functools

import jax
import jax.numpy as jnp
from jax import lax
from jax.experimental import pallas as pl
from jax.experimental.pallas import tpu as pltpu
from jax.experimental.pallas import tpu_sc as plsc

N = 10000
E = 320000
D = 128

NC = 2
NS = 16
NW = NC * NS
PER_TILE = E // NW
CH = 80
N_CHUNK = PER_TILE // CH
NP = 10112
ROWS_PER_TILE = NP // NS

_MESH = plsc.VectorSubcoreMesh(core_axis_name="c", subcore_axis_name="s")

DEG_W = 128


@functools.partial(
    pl.kernel,
    out_type=jax.ShapeDtypeStruct((NC, NP, DEG_W), jnp.float32),
    mesh=_MESH,
    scratch_types=[
        pltpu.VMEM((CH,), jnp.int32),
        pltpu.VMEM((CH, DEG_W), jnp.float32),
        pltpu.VMEM_SHARED((NP, DEG_W), jnp.float32),
    ],
)
def _sc_degree(dst_hbm, ones_hbm, zeros_hbm, out_hbm, idx_v, ones_v, acc_s):
    c = lax.axis_index("c")
    s = lax.axis_index("s")
    wid = s * NC + c
    pltpu.sync_copy(zeros_hbm, acc_s.at[pl.ds(s * ROWS_PER_TILE, ROWS_PER_TILE)])
    pltpu.sync_copy(ones_hbm, ones_v)
    plsc.subcore_barrier()
    base = wid * PER_TILE

    def body(i, carry):
        off = base + i * CH
        pltpu.sync_copy(dst_hbm.at[pl.ds(off, CH)], idx_v)
        pltpu.sync_copy(ones_v, acc_s.at[idx_v], add=True)
        return carry

    lax.fori_loop(0, N_CHUNK, body, 0)
    plsc.subcore_barrier()
    pltpu.sync_copy(
        acc_s.at[pl.ds(s * ROWS_PER_TILE, ROWS_PER_TILE)],
        out_hbm.at[c, pl.ds(s * ROWS_PER_TILE, ROWS_PER_TILE)],
    )


@functools.partial(
    pl.kernel,
    out_type=jax.ShapeDtypeStruct((NC, NP, D), jnp.float32),
    mesh=_MESH,
    scratch_types=[
        pltpu.VMEM((2, CH), jnp.int32),
        pltpu.VMEM((2, CH), jnp.int32),
        pltpu.VMEM((2, CH, D), jnp.float32),
        pltpu.VMEM_SHARED((NP, D), jnp.float32),
        pltpu.SemaphoreType.DMA,
        pltpu.SemaphoreType.DMA,
    ],
)
def _sc_edge_pass(z_hbm, src_hbm, dst_hbm, zeros_hbm, out_hbm,
                  src_v, dst_v, rows_v, acc_s, semA, semB):
    c = lax.axis_index("c")
    s = lax.axis_index("s")
    wid = s * NC + c
    pltpu.sync_copy(zeros_hbm, acc_s.at[pl.ds(s * ROWS_PER_TILE, ROWS_PER_TILE)])
    plsc.subcore_barrier()
    base = wid * PER_TILE

    def load_idx(i, p):
        off = base + i * CH
        pltpu.sync_copy(src_hbm.at[pl.ds(off, CH)], src_v.at[p])
        pltpu.sync_copy(dst_hbm.at[pl.ds(off, CH)], dst_v.at[p])

    def gather_start(p, sem):
        pltpu.async_copy(z_hbm.at[src_v.at[p]], rows_v.at[p], sem)

    def gather_wait(p, sem):
        pltpu.make_async_copy(z_hbm.at[src_v.at[p]], rows_v.at[p], sem).wait()

    def scatter(p):
        pltpu.sync_copy(rows_v.at[p], acc_s.at[dst_v.at[p]], add=True)

    load_idx(0, 0)
    gather_start(0, semA)

    def body(i, carry):
        @pl.when(i % 2 == 0)
        def _():
            load_idx(i + 1, 1)
            gather_start(1, semB)
            gather_wait(0, semA)
            scatter(0)

        @pl.when(i % 2 == 1)
        def _():
            load_idx(i + 1, 0)
            gather_start(0, semA)
            gather_wait(1, semB)
            scatter(1)

        return carry

    lax.fori_loop(0, N_CHUNK - 1, body, 0)
    gather_wait(0, semA)
    scatter(0)
    plsc.subcore_barrier()
    pltpu.sync_copy(
        acc_s.at[pl.ds(s * ROWS_PER_TILE, ROWS_PER_TILE)],
        out_hbm.at[c, pl.ds(s * ROWS_PER_TILE, ROWS_PER_TILE)],
    )


L = 16
NVEC = PER_TILE // L
LCAP = PER_TILE + 2 * L
TRASH = LCAP - 1


@functools.partial(
    pl.kernel,
    out_type=jax.ShapeDtypeStruct((NC, NS, 8, D), jnp.float32),
    mesh=_MESH,
    compiler_params=pltpu.CompilerParams(needs_layout_passes=False),
    scratch_types=[
        pltpu.VMEM((PER_TILE,), jnp.int32),
        pltpu.VMEM((PER_TILE,), jnp.int32),
        pltpu.VMEM((LCAP,), jnp.int32),
        pltpu.VMEM((LCAP,), jnp.int32),
        pltpu.VMEM((2, L), jnp.int32),
        pltpu.VMEM((L,), jnp.int32),
        pltpu.VMEM((L, D), jnp.float32),
        pltpu.VMEM((L, D), jnp.float32),
        pltpu.VMEM((8, D), jnp.float32),
        pltpu.SemaphoreType.DMA,
    ],
)
def _sc_filter(src_hbm, dst_hbm, ops_hbm, q1_hbm, q2_hbm, out_hbm,
               dst_b, src_b, list1, list2, ops_v, pad_v, rows1, rows2, uacc, sem):
    c = lax.axis_index("c")
    s = lax.axis_index("s")
    wid = s * NC + c
    base = wid * PER_TILE
    pltpu.sync_copy(dst_hbm.at[pl.ds(base, PER_TILE)], dst_b)
    pltpu.sync_copy(src_hbm.at[pl.ds(base, PER_TILE)], src_b)
    pltpu.sync_copy(ops_hbm, ops_v)
    op1v = ops_v[0, :]
    op2v = ops_v[1, :]
    z16 = jnp.zeros((L,), jnp.float32)
    for r in range(8):
        for j in range(D // L):
            uacc[r, pl.ds(j * L, L)] = z16

    def scan_body(i, carry):
        cnt1, cnt2 = carry
        dv = dst_b[pl.ds(i * L, L)]
        sv = src_b[pl.ds(i * L, L)]
        m1 = dv == op1v
        m2 = dv == op2v
        one16 = jnp.ones((L,), jnp.int32)
        trash16 = jnp.full((L,), TRASH, jnp.int32)
        pos1 = plsc.cumsum(m1.astype(jnp.int32))
        pos2 = plsc.cumsum(m2.astype(jnp.int32))
        c1v = jnp.full((L,), cnt1, jnp.int32)
        c2v = jnp.full((L,), cnt2, jnp.int32)
        idx1 = jnp.where(m1, c1v + pos1 - one16, trash16)
        idx2 = jnp.where(m2, c2v + pos2 - one16, trash16)
        plsc.store_scatter(list1, [idx1], sv)
        plsc.store_scatter(list2, [idx2], sv)
        return cnt1 + jnp.max(pos1), cnt2 + jnp.max(pos2)

    cnt1, cnt2 = lax.fori_loop(0, NVEC, scan_body, (jnp.int32(0), jnp.int32(0)))

    zi16 = jnp.zeros((L,), jnp.int32)
    pad_v[...] = zi16

    def accumulate(lst, cnt, r1, r2):
        lst[pl.ds(cnt, L)] = zi16
        nch = (cnt + (L - 1)) // L

        def body(k, carry):
            ids = lst.at[pl.ds(k * L, L)]
            pltpu.async_copy(q1_hbm.at[ids], rows1, sem).wait()
            pltpu.async_copy(q2_hbm.at[ids], rows2, sem).wait()
            for j in range(D // L):
                sl = pl.ds(j * L, L)
                a1 = uacc[r1, sl]
                a2 = uacc[r2, sl]
                for t in range(L):
                    a1 = a1 + rows1[t, sl]
                    a2 = a2 + rows2[t, sl]
                uacc[r1, sl] = a1
                uacc[r2, sl] = a2
            return carry

        lax.fori_loop(0, nch, body, 0)
        nspv = jnp.full((L,), (nch * L - cnt).astype(jnp.float32))
        pltpu.async_copy(q1_hbm.at[pad_v], rows1, sem).wait()
        pltpu.async_copy(q2_hbm.at[pad_v], rows2, sem).wait()
        for j in range(D // L):
            sl = pl.ds(j * L, L)
            uacc[r1, sl] = uacc[r1, sl] - nspv * rows1[0, sl]
            uacc[r2, sl] = uacc[r2, sl] - nspv * rows2[0, sl]

    accumulate(list1, cnt1, 0, 2)
    accumulate(list2, cnt2, 1, 3)
    pltpu.sync_copy(uacc, out_hbm.at[c, s])


ROW_BLK = 400
N_BLK = N // ROW_BLK


def _tc_dinv(deg_part):
    def body(p_ref, o_ref):
        deg = p_ref[0, :, 0:1] + p_ref[1, :, 0:1] + 1.0
        o_ref[...] = lax.rsqrt(deg)

    return pl.pallas_call(
        body,
        grid=(N_BLK,),
        in_specs=[pl.BlockSpec((NC, ROW_BLK, DEG_W), lambda i: (0, i, 0))],
        out_specs=pl.BlockSpec((ROW_BLK, 1), lambda i: (i, 0)),
        out_shape=jax.ShapeDtypeStruct((N, 1), jnp.float32),
    )(deg_part)


def _tc_pre(x, dinv, W):
    def body(x_ref, d_ref, w_ref, o_ref):
        o_ref[...] = jnp.dot(d_ref[...] * x_ref[...], w_ref[...],
                             preferred_element_type=jnp.float32)

    return pl.pallas_call(
        body,
        grid=(N_BLK,),
        in_specs=[
            pl.BlockSpec((ROW_BLK, D), lambda i: (i, 0)),
            pl.BlockSpec((ROW_BLK, 1), lambda i: (i, 0)),
            pl.BlockSpec((D, D), lambda i: (0, 0)),
        ],
        out_specs=pl.BlockSpec((ROW_BLK, D), lambda i: (i, 0)),
        out_shape=jax.ShapeDtypeStruct((N, D), jnp.float32),
    )(x, dinv, W)


def _tc_q(part, z, dinv, b):
    def body(p_ref, z_ref, d_ref, b_ref, o_ref):
        agg = p_ref[0] + p_ref[1] + z_ref[...]
        h = jnp.maximum(d_ref[...] * agg + b_ref[...], 0.0)
        o_ref[...] = d_ref[...] * h

    return pl.pallas_call(
        body,
        grid=(N_BLK,),
        in_specs=[
            pl.BlockSpec((NC, ROW_BLK, D), lambda i: (0, i, 0)),
            pl.BlockSpec((ROW_BLK, D), lambda i: (i, 0)),
            pl.BlockSpec((ROW_BLK, 1), lambda i: (i, 0)),
            pl.BlockSpec((1, D), lambda i: (0, 0)),
        ],
        out_specs=pl.BlockSpec((ROW_BLK, D), lambda i: (i, 0)),
        out_shape=jax.ShapeDtypeStruct((N, D), jnp.float32),
    )(part, z, dinv, b)


def _tc_finish(slabs, q1, q2, dinv, W1b, b1b, W2b, b2b, ops):
    def body(ops_ref, sl_ref, q1_ref, q2_ref, d_ref, w1_ref, b1_ref,
             w2_ref, b2_ref, o_ref):
        u = jnp.sum(sl_ref[...], axis=0)
        o1 = ops_ref[0]
        o2 = ops_ref[1]
        u11 = u[0:1] + q1_ref[pl.ds(o1, 1), :]
        u12 = u[1:2] + q1_ref[pl.ds(o2, 1), :]
        u21 = u[2:3] + q2_ref[pl.ds(o1, 1), :]
        u22 = u[3:4] + q2_ref[pl.ds(o2, 1), :]
        d1 = d_ref[pl.ds(o1, 1), :]
        d2 = d_ref[pl.ds(o2, 1), :]
        a = jnp.dot(jnp.concatenate([u11, u12], axis=0), w1_ref[...],
                    preferred_element_type=jnp.float32)
        b = jnp.dot(jnp.concatenate([u21, u22], axis=0), w2_ref[...],
                    preferred_element_type=jnp.float32)
        y11 = d1 * a[0:1] + b1_ref[...]
        y12 = d2 * a[1:2] + b1_ref[...]
        y21 = d1 * b[0:1] + b2_ref[...]
        y22 = d2 * b[1:2] + b2_ref[...]
        o_ref[0] = jnp.sum(y11 * y12)
        o_ref[1] = jnp.sum(y21 * y22)

    return pl.pallas_call(
        body,
        in_specs=[
            pl.BlockSpec(memory_space=pltpu.SMEM),
            pl.BlockSpec(memory_space=pltpu.VMEM),
            pl.BlockSpec(memory_space=pltpu.VMEM),
            pl.BlockSpec(memory_space=pltpu.VMEM),
            pl.BlockSpec(memory_space=pltpu.VMEM),
            pl.BlockSpec(memory_space=pltpu.VMEM),
            pl.BlockSpec(memory_space=pltpu.VMEM),
            pl.BlockSpec(memory_space=pltpu.VMEM),
            pl.BlockSpec(memory_space=pltpu.VMEM),
        ],
        out_specs=pl.BlockSpec(memory_space=pltpu.SMEM),
        out_shape=jax.ShapeDtypeStruct((2,), jnp.float32),
    )(ops, slabs, q1, q2, dinv, W1b, b1b, W2b, b2b)


def kernel(x, edge_index, op1, op2, W1a, b1a, W1b, b1b, W2a, b2a, W2b, b2b):
    src = edge_index[0]
    dst = edge_index[1]
    ones_deg = jnp.ones((CH, DEG_W), jnp.float32)
    zeros_deg = jnp.zeros((ROWS_PER_TILE, DEG_W), jnp.float32)
    zeros_row = jnp.zeros((ROWS_PER_TILE, D), jnp.float32)
    ops = jnp.stack([op1, op2]).astype(jnp.int32)
    ops16 = jnp.broadcast_to(ops[:, None], (2, L))

    deg_part = _sc_degree(dst, ones_deg, zeros_deg)
    dinv = _tc_dinv(deg_part)

    z1 = _tc_pre(x, dinv, W1a)
    z2 = _tc_pre(x, dinv, W2a)
    p1 = _sc_edge_pass(z1, src, dst, zeros_row)
    p2 = _sc_edge_pass(z2, src, dst, zeros_row)
    q1 = _tc_q(p1, z1, dinv, b1a.reshape(1, D))
    q2 = _tc_q(p2, z2, dinv, b2a.reshape(1, D))

    slabs = _sc_filter(src, dst, ops16, q1, q2)
    slabs = slabs.reshape(NC * NS, 8, D)
    return _tc_finish(slabs, q1, q2, dinv, W1b, b1b.reshape(1, D),
                      W2b, b2b.reshape(1, D), ops)


# speedup vs baseline: 22.8632x; 1.2115x over previous
import functools

import jax
import jax.numpy as jnp
from jax import lax
from jax.experimental import pallas as pl
from jax.experimental.pallas import tpu as pltpu
from jax.experimental.pallas import tpu_sc as plsc

N = 10000
E = 320000
D = 128

NC = 2
NS = 16
NW = NC * NS
PER_TILE = E // NW
CH = 80
N_CHUNK = PER_TILE // CH
NP = 10112
ROWS_PER_TILE = NP // NS

_MESH = plsc.VectorSubcoreMesh(core_axis_name="c", subcore_axis_name="s")

DEG_W = 128


@functools.partial(
    pl.kernel,
    out_type=jax.ShapeDtypeStruct((NC, NP, DEG_W), jnp.float32),
    mesh=_MESH,
    scratch_types=[
        pltpu.VMEM((CH,), jnp.int32),
        pltpu.VMEM((CH, DEG_W), jnp.float32),
        pltpu.VMEM_SHARED((NP, DEG_W), jnp.float32),
    ],
)
def _sc_degree(dst_hbm, ones_hbm, zeros_hbm, out_hbm, idx_v, ones_v, acc_s):
    c = lax.axis_index("c")
    s = lax.axis_index("s")
    wid = s * NC + c
    pltpu.sync_copy(zeros_hbm, acc_s.at[pl.ds(s * ROWS_PER_TILE, ROWS_PER_TILE)])
    pltpu.sync_copy(ones_hbm, ones_v)
    plsc.subcore_barrier()
    base = wid * PER_TILE

    def body(i, carry):
        off = base + i * CH
        pltpu.sync_copy(dst_hbm.at[pl.ds(off, CH)], idx_v)
        pltpu.sync_copy(ones_v, acc_s.at[idx_v], add=True)
        return carry

    lax.fori_loop(0, N_CHUNK, body, 0)
    plsc.subcore_barrier()
    pltpu.sync_copy(
        acc_s.at[pl.ds(s * ROWS_PER_TILE, ROWS_PER_TILE)],
        out_hbm.at[c, pl.ds(s * ROWS_PER_TILE, ROWS_PER_TILE)],
    )


@functools.partial(
    pl.kernel,
    out_type=jax.ShapeDtypeStruct((NC, NP, D), jnp.float32),
    mesh=_MESH,
    scratch_types=[
        pltpu.VMEM((PER_TILE,), jnp.int32),
        pltpu.VMEM((N_CHUNK, CH), jnp.int32),
        pltpu.VMEM((2, CH, D), jnp.float32),
        pltpu.VMEM_SHARED((NP, D), jnp.float32),
        pltpu.SemaphoreType.DMA,
        pltpu.SemaphoreType.DMA,
    ],
)
def _sc_edge_pass(z_hbm, src_hbm, dst_hbm, zeros_hbm, out_hbm,
                  src_v, dst_v, rows_v, acc_s, semA, semB):
    c = lax.axis_index("c")
    s = lax.axis_index("s")
    wid = s * NC + c
    pltpu.sync_copy(zeros_hbm, acc_s.at[pl.ds(s * ROWS_PER_TILE, ROWS_PER_TILE)])
    pltpu.sync_copy(src_hbm.at[pl.ds(wid * PER_TILE, PER_TILE)], src_v)
    pltpu.sync_copy(dst_hbm.at[wid], dst_v)
    plsc.subcore_barrier()

    def gather_start(i, p, sem):
        pltpu.async_copy(z_hbm.at[src_v.at[pl.ds(i * CH, CH)]], rows_v.at[p], sem)

    def gather_wait(i, p, sem):
        pltpu.make_async_copy(
            z_hbm.at[src_v.at[pl.ds(i * CH, CH)]], rows_v.at[p], sem).wait()

    def scatter(i, p):
        pltpu.sync_copy(rows_v.at[p], acc_s.at[dst_v.at[i]], add=True)

    gather_start(0, 0, semA)

    def body(i, carry):
        @pl.when(i % 2 == 0)
        def _():
            gather_start(i + 1, 1, semB)
            gather_wait(i, 0, semA)
            scatter(i, 0)

        @pl.when(i % 2 == 1)
        def _():
            gather_start(i + 1, 0, semA)
            gather_wait(i, 1, semB)
            scatter(i, 1)

        return carry

    lax.fori_loop(0, N_CHUNK - 1, body, 0)
    gather_wait(N_CHUNK - 1, 0, semA)
    scatter(N_CHUNK - 1, 0)
    plsc.subcore_barrier()
    pltpu.sync_copy(
        acc_s.at[pl.ds(s * ROWS_PER_TILE, ROWS_PER_TILE)],
        out_hbm.at[c, pl.ds(s * ROWS_PER_TILE, ROWS_PER_TILE)],
    )


L = 16
NVEC = PER_TILE // L
LCAP = PER_TILE + 2 * L
TRASH = LCAP - 1


@functools.partial(
    pl.kernel,
    out_type=jax.ShapeDtypeStruct((NC, NS, 8, D), jnp.float32),
    mesh=_MESH,
    compiler_params=pltpu.CompilerParams(needs_layout_passes=False),
    scratch_types=[
        pltpu.VMEM((PER_TILE,), jnp.int32),
        pltpu.VMEM((PER_TILE,), jnp.int32),
        pltpu.VMEM((LCAP,), jnp.int32),
        pltpu.VMEM((LCAP,), jnp.int32),
        pltpu.VMEM((2, L), jnp.int32),
        pltpu.VMEM((L,), jnp.int32),
        pltpu.VMEM((L, D), jnp.float32),
        pltpu.VMEM((L, D), jnp.float32),
        pltpu.VMEM((8, D), jnp.float32),
        pltpu.SemaphoreType.DMA,
    ],
)
def _sc_filter(src_hbm, dst_hbm, ops_hbm, q1_hbm, q2_hbm, out_hbm,
               dst_b, src_b, list1, list2, ops_v, pad_v, rows1, rows2, uacc, sem):
    c = lax.axis_index("c")
    s = lax.axis_index("s")
    wid = s * NC + c
    base = wid * PER_TILE
    pltpu.sync_copy(dst_hbm.at[pl.ds(base, PER_TILE)], dst_b)
    pltpu.sync_copy(src_hbm.at[pl.ds(base, PER_TILE)], src_b)
    pltpu.sync_copy(ops_hbm, ops_v)
    op1v = ops_v[0, :]
    op2v = ops_v[1, :]
    z16 = jnp.zeros((L,), jnp.float32)
    for r in range(8):
        for j in range(D // L):
            uacc[r, pl.ds(j * L, L)] = z16

    def scan_body(i, carry):
        dv = dst_b[pl.ds(i * L, L)]
        m1 = dv == op1v
        m2 = dv == op2v
        pc = plsc.all_reduce_population_count(m1 | m2)

        def slow(c1, c2):
            sv = src_b[pl.ds(i * L, L)]
            one16 = jnp.ones((L,), jnp.int32)
            trash16 = jnp.full((L,), TRASH, jnp.int32)
            pos1 = plsc.cumsum(m1.astype(jnp.int32))
            pos2 = plsc.cumsum(m2.astype(jnp.int32))
            c1v = jnp.full((L,), c1, jnp.int32)
            c2v = jnp.full((L,), c2, jnp.int32)
            idx1 = jnp.where(m1, c1v + pos1 - one16, trash16)
            idx2 = jnp.where(m2, c2v + pos2 - one16, trash16)
            plsc.store_scatter(list1, [idx1], sv)
            plsc.store_scatter(list2, [idx2], sv)
            return c1 + jnp.max(pos1), c2 + jnp.max(pos2)

        def fast(c1, c2):
            return c1, c2

        return lax.cond(pc[0] > 0, slow, fast, *carry)

    cnt1, cnt2 = lax.fori_loop(0, NVEC, scan_body, (jnp.int32(0), jnp.int32(0)))

    zi16 = jnp.zeros((L,), jnp.int32)
    pad_v[...] = zi16

    def accumulate(lst, cnt, r1, r2):
        lst[pl.ds(cnt, L)] = zi16
        nch = (cnt + (L - 1)) // L

        def body(k, carry):
            ids = lst.at[pl.ds(k * L, L)]
            pltpu.async_copy(q1_hbm.at[ids], rows1, sem).wait()
            pltpu.async_copy(q2_hbm.at[ids], rows2, sem).wait()
            for j in range(D // L):
                sl = pl.ds(j * L, L)
                a1 = uacc[r1, sl]
                a2 = uacc[r2, sl]
                for t in range(L):
                    a1 = a1 + rows1[t, sl]
                    a2 = a2 + rows2[t, sl]
                uacc[r1, sl] = a1
                uacc[r2, sl] = a2
            return carry

        lax.fori_loop(0, nch, body, 0)
        nspv = jnp.full((L,), (nch * L - cnt).astype(jnp.float32))
        pltpu.async_copy(q1_hbm.at[pad_v], rows1, sem).wait()
        pltpu.async_copy(q2_hbm.at[pad_v], rows2, sem).wait()
        for j in range(D // L):
            sl = pl.ds(j * L, L)
            uacc[r1, sl] = uacc[r1, sl] - nspv * rows1[0, sl]
            uacc[r2, sl] = uacc[r2, sl] - nspv * rows2[0, sl]

    accumulate(list1, cnt1, 0, 2)
    accumulate(list2, cnt2, 1, 3)
    pltpu.sync_copy(uacc, out_hbm.at[c, s])


ROW_BLK = 400
N_BLK = N // ROW_BLK


def _tc_dinv(deg_part):
    def body(p_ref, o_ref):
        deg = p_ref[0, :, 0:1] + p_ref[1, :, 0:1] + 1.0
        o_ref[...] = lax.rsqrt(deg)

    return pl.pallas_call(
        body,
        grid=(N_BLK,),
        in_specs=[pl.BlockSpec((NC, ROW_BLK, DEG_W), lambda i: (0, i, 0))],
        out_specs=pl.BlockSpec((ROW_BLK, 1), lambda i: (i, 0)),
        out_shape=jax.ShapeDtypeStruct((N, 1), jnp.float32),
    )(deg_part)


def _tc_pre(x, dinv, W):
    def body(x_ref, d_ref, w_ref, o_ref):
        o_ref[...] = jnp.dot(d_ref[...] * x_ref[...], w_ref[...],
                             preferred_element_type=jnp.float32)

    return pl.pallas_call(
        body,
        grid=(N_BLK,),
        in_specs=[
            pl.BlockSpec((ROW_BLK, D), lambda i: (i, 0)),
            pl.BlockSpec((ROW_BLK, 1), lambda i: (i, 0)),
            pl.BlockSpec((D, D), lambda i: (0, 0)),
        ],
        out_specs=pl.BlockSpec((ROW_BLK, D), lambda i: (i, 0)),
        out_shape=jax.ShapeDtypeStruct((N, D), jnp.float32),
    )(x, dinv, W)


def _tc_q(part, z, dinv, b):
    def body(p_ref, z_ref, d_ref, b_ref, o_ref):
        agg = p_ref[0] + p_ref[1] + z_ref[...]
        h = jnp.maximum(d_ref[...] * agg + b_ref[...], 0.0)
        o_ref[...] = d_ref[...] * h

    return pl.pallas_call(
        body,
        grid=(N_BLK,),
        in_specs=[
            pl.BlockSpec((NC, ROW_BLK, D), lambda i: (0, i, 0)),
            pl.BlockSpec((ROW_BLK, D), lambda i: (i, 0)),
            pl.BlockSpec((ROW_BLK, 1), lambda i: (i, 0)),
            pl.BlockSpec((1, D), lambda i: (0, 0)),
        ],
        out_specs=pl.BlockSpec((ROW_BLK, D), lambda i: (i, 0)),
        out_shape=jax.ShapeDtypeStruct((N, D), jnp.float32),
    )(part, z, dinv, b)


def _tc_finish(slabs, q1, q2, dinv, W1b, b1b, W2b, b2b, ops):
    def body(ops_ref, sl_ref, q1_ref, q2_ref, d_ref, w1_ref, b1_ref,
             w2_ref, b2_ref, o_ref):
        u = jnp.sum(sl_ref[...], axis=0)
        o1 = ops_ref[0]
        o2 = ops_ref[1]
        u11 = u[0:1] + q1_ref[pl.ds(o1, 1), :]
        u12 = u[1:2] + q1_ref[pl.ds(o2, 1), :]
        u21 = u[2:3] + q2_ref[pl.ds(o1, 1), :]
        u22 = u[3:4] + q2_ref[pl.ds(o2, 1), :]
        d1 = d_ref[pl.ds(o1, 1), :]
        d2 = d_ref[pl.ds(o2, 1), :]
        a = jnp.dot(jnp.concatenate([u11, u12], axis=0), w1_ref[...],
                    preferred_element_type=jnp.float32)
        b = jnp.dot(jnp.concatenate([u21, u22], axis=0), w2_ref[...],
                    preferred_element_type=jnp.float32)
        y11 = d1 * a[0:1] + b1_ref[...]
        y12 = d2 * a[1:2] + b1_ref[...]
        y21 = d1 * b[0:1] + b2_ref[...]
        y22 = d2 * b[1:2] + b2_ref[...]
        o_ref[0] = jnp.sum(y11 * y12)
        o_ref[1] = jnp.sum(y21 * y22)

    return pl.pallas_call(
        body,
        in_specs=[
            pl.BlockSpec(memory_space=pltpu.SMEM),
            pl.BlockSpec(memory_space=pltpu.VMEM),
            pl.BlockSpec(memory_space=pltpu.VMEM),
            pl.BlockSpec(memory_space=pltpu.VMEM),
            pl.BlockSpec(memory_space=pltpu.VMEM),
            pl.BlockSpec(memory_space=pltpu.VMEM),
            pl.BlockSpec(memory_space=pltpu.VMEM),
            pl.BlockSpec(memory_space=pltpu.VMEM),
            pl.BlockSpec(memory_space=pltpu.VMEM),
        ],
        out_specs=pl.BlockSpec(memory_space=pltpu.SMEM),
        out_shape=jax.ShapeDtypeStruct((2,), jnp.float32),
    )(ops, slabs, q1, q2, dinv, W1b, b1b, W2b, b2b)


def kernel(x, edge_index, op1, op2, W1a, b1a, W1b, b1b, W2a, b2a, W2b, b2b):
    src = edge_index[0]
    dst = edge_index[1]
    ones_deg = jnp.ones((CH, DEG_W), jnp.float32)
    zeros_deg = jnp.zeros((ROWS_PER_TILE, DEG_W), jnp.float32)
    zeros_row = jnp.zeros((ROWS_PER_TILE, D), jnp.float32)
    ops = jnp.stack([op1, op2]).astype(jnp.int32)
    ops16 = jnp.broadcast_to(ops[:, None], (2, L))

    deg_part = _sc_degree(dst, ones_deg, zeros_deg)
    dinv = _tc_dinv(deg_part)

    dst_t = dst.reshape(NW, N_CHUNK, CH)
    z1 = _tc_pre(x, dinv, W1a)
    z2 = _tc_pre(x, dinv, W2a)
    p1 = _sc_edge_pass(z1, src, dst_t, zeros_row)
    p2 = _sc_edge_pass(z2, src, dst_t, zeros_row)
    q1 = _tc_q(p1, z1, dinv, b1a.reshape(1, D))
    q2 = _tc_q(p2, z2, dinv, b2a.reshape(1, D))

    slabs = _sc_filter(src, dst, ops16, q1, q2)
    slabs = slabs.reshape(NC * NS, 8, D)
    return _tc_finish(slabs, q1, q2, dinv, W1b, b1b.reshape(1, D),
                      W2b, b2b.reshape(1, D), ops)


# speedup vs baseline: 23.1111x; 1.0108x over previous
import functools

import jax
import jax.numpy as jnp
from jax import lax
from jax.experimental import pallas as pl
from jax.experimental.pallas import tpu as pltpu
from jax.experimental.pallas import tpu_sc as plsc

N = 10000
E = 320000
D = 128

NC = 2
NS = 16
NW = NC * NS
PER_TILE = E // NW
CH = 80
N_CHUNK = PER_TILE // CH
NP = 10112
ROWS_PER_TILE = NP // NS

_MESH = plsc.VectorSubcoreMesh(core_axis_name="c", subcore_axis_name="s")

L = 16
NVEC = PER_TILE // L
NPVEC = NP // L


@functools.partial(
    pl.kernel,
    out_type=jax.ShapeDtypeStruct((NW, NP), jnp.float32),
    mesh=_MESH,
    compiler_params=pltpu.CompilerParams(needs_layout_passes=False),
    scratch_types=[
        pltpu.VMEM((PER_TILE,), jnp.int32),
        pltpu.VMEM((NP,), jnp.float32),
    ],
)
def _sc_degree(dst_hbm, out_hbm, dst_b, hist):
    c = lax.axis_index("c")
    s = lax.axis_index("s")
    wid = s * NC + c
    pltpu.sync_copy(dst_hbm.at[pl.ds(wid * PER_TILE, PER_TILE)], dst_b)
    z16 = jnp.zeros((L,), jnp.float32)
    one16 = jnp.ones((L,), jnp.float32)

    def zbody(i, carry):
        hist[pl.ds(i * L, L)] = z16
        return carry

    lax.fori_loop(0, NPVEC, zbody, 0)

    def body(i, carry):
        idx = dst_b[pl.ds(i * L, L)]
        plsc.addupdate_scatter(hist, [idx], one16)
        return carry

    lax.fori_loop(0, NVEC, body, 0)
    pltpu.sync_copy(hist, out_hbm.at[wid])


@functools.partial(
    pl.kernel,
    out_type=jax.ShapeDtypeStruct((NC, NP, D), jnp.float32),
    mesh=_MESH,
    scratch_types=[
        pltpu.VMEM((PER_TILE,), jnp.int32),
        pltpu.VMEM((N_CHUNK, CH), jnp.int32),
        pltpu.VMEM((2, CH, D), jnp.float32),
        pltpu.VMEM_SHARED((NP, D), jnp.float32),
        pltpu.SemaphoreType.DMA,
        pltpu.SemaphoreType.DMA,
    ],
)
def _sc_edge_pass(z_hbm, src_hbm, dst_hbm, zeros_hbm, out_hbm,
                  src_v, dst_v, rows_v, acc_s, semA, semB):
    c = lax.axis_index("c")
    s = lax.axis_index("s")
    wid = s * NC + c
    pltpu.sync_copy(zeros_hbm, acc_s.at[pl.ds(s * ROWS_PER_TILE, ROWS_PER_TILE)])
    pltpu.sync_copy(src_hbm.at[pl.ds(wid * PER_TILE, PER_TILE)], src_v)
    pltpu.sync_copy(dst_hbm.at[wid], dst_v)
    plsc.subcore_barrier()

    def gather_start(i, p, sem):
        pltpu.async_copy(z_hbm.at[src_v.at[pl.ds(i * CH, CH)]], rows_v.at[p], sem)

    def gather_wait(i, p, sem):
        pltpu.make_async_copy(
            z_hbm.at[src_v.at[pl.ds(i * CH, CH)]], rows_v.at[p], sem).wait()

    def scatter(i, p):
        pltpu.sync_copy(rows_v.at[p], acc_s.at[dst_v.at[i]], add=True)

    gather_start(0, 0, semA)

    def body(i, carry):
        @pl.when(i % 2 == 0)
        def _():
            gather_start(i + 1, 1, semB)
            gather_wait(i, 0, semA)
            scatter(i, 0)

        @pl.when(i % 2 == 1)
        def _():
            gather_start(i + 1, 0, semA)
            gather_wait(i, 1, semB)
            scatter(i, 1)

        return carry

    lax.fori_loop(0, N_CHUNK - 1, body, 0)
    gather_wait(N_CHUNK - 1, 0, semA)
    scatter(N_CHUNK - 1, 0)
    plsc.subcore_barrier()
    pltpu.sync_copy(
        acc_s.at[pl.ds(s * ROWS_PER_TILE, ROWS_PER_TILE)],
        out_hbm.at[c, pl.ds(s * ROWS_PER_TILE, ROWS_PER_TILE)],
    )


LCAP = PER_TILE + 2 * L
TRASH = LCAP - 1


@functools.partial(
    pl.kernel,
    out_type=jax.ShapeDtypeStruct((NC, NS, 8, D), jnp.float32),
    mesh=_MESH,
    compiler_params=pltpu.CompilerParams(needs_layout_passes=False),
    scratch_types=[
        pltpu.VMEM((PER_TILE,), jnp.int32),
        pltpu.VMEM((PER_TILE,), jnp.int32),
        pltpu.VMEM((LCAP,), jnp.int32),
        pltpu.VMEM((LCAP,), jnp.int32),
        pltpu.VMEM((2, L), jnp.int32),
        pltpu.VMEM((L,), jnp.int32),
        pltpu.VMEM((L, D), jnp.float32),
        pltpu.VMEM((L, D), jnp.float32),
        pltpu.VMEM((8, D), jnp.float32),
        pltpu.SemaphoreType.DMA,
    ],
)
def _sc_filter(src_hbm, dst_hbm, ops_hbm, q1_hbm, q2_hbm, out_hbm,
               dst_b, src_b, list1, list2, ops_v, pad_v, rows1, rows2, uacc, sem):
    c = lax.axis_index("c")
    s = lax.axis_index("s")
    wid = s * NC + c
    base = wid * PER_TILE
    pltpu.sync_copy(dst_hbm.at[pl.ds(base, PER_TILE)], dst_b)
    pltpu.sync_copy(src_hbm.at[pl.ds(base, PER_TILE)], src_b)
    pltpu.sync_copy(ops_hbm, ops_v)
    op1v = ops_v[0, :]
    op2v = ops_v[1, :]
    z16 = jnp.zeros((L,), jnp.float32)
    for r in range(8):
        for j in range(D // L):
            uacc[r, pl.ds(j * L, L)] = z16

    def scan_body(i, carry):
        dv = dst_b[pl.ds(i * L, L)]
        m1 = dv == op1v
        m2 = dv == op2v
        pc = plsc.all_reduce_population_count(m1 | m2)

        def slow(c1, c2):
            sv = src_b[pl.ds(i * L, L)]
            one16 = jnp.ones((L,), jnp.int32)
            trash16 = jnp.full((L,), TRASH, jnp.int32)
            pos1 = plsc.cumsum(m1.astype(jnp.int32))
            pos2 = plsc.cumsum(m2.astype(jnp.int32))
            c1v = jnp.full((L,), c1, jnp.int32)
            c2v = jnp.full((L,), c2, jnp.int32)
            idx1 = jnp.where(m1, c1v + pos1 - one16, trash16)
            idx2 = jnp.where(m2, c2v + pos2 - one16, trash16)
            plsc.store_scatter(list1, [idx1], sv)
            plsc.store_scatter(list2, [idx2], sv)
            return c1 + jnp.max(pos1), c2 + jnp.max(pos2)

        def fast(c1, c2):
            return c1, c2

        return lax.cond(pc[0] > 0, slow, fast, *carry)

    cnt1, cnt2 = lax.fori_loop(0, NVEC, scan_body, (jnp.int32(0), jnp.int32(0)))

    zi16 = jnp.zeros((L,), jnp.int32)
    pad_v[...] = zi16

    def accumulate(lst, cnt, r1, r2):
        lst[pl.ds(cnt, L)] = zi16
        nch = (cnt + (L - 1)) // L

        def body(k, carry):
            ids = lst.at[pl.ds(k * L, L)]
            pltpu.async_copy(q1_hbm.at[ids], rows1, sem).wait()
            pltpu.async_copy(q2_hbm.at[ids], rows2, sem).wait()
            for j in range(D // L):
                sl = pl.ds(j * L, L)
                a1 = uacc[r1, sl]
                a2 = uacc[r2, sl]
                for t in range(L):
                    a1 = a1 + rows1[t, sl]
                    a2 = a2 + rows2[t, sl]
                uacc[r1, sl] = a1
                uacc[r2, sl] = a2
            return carry

        lax.fori_loop(0, nch, body, 0)
        nspv = jnp.full((L,), (nch * L - cnt).astype(jnp.float32))
        pltpu.async_copy(q1_hbm.at[pad_v], rows1, sem).wait()
        pltpu.async_copy(q2_hbm.at[pad_v], rows2, sem).wait()
        for j in range(D // L):
            sl = pl.ds(j * L, L)
            uacc[r1, sl] = uacc[r1, sl] - nspv * rows1[0, sl]
            uacc[r2, sl] = uacc[r2, sl] - nspv * rows2[0, sl]

    accumulate(list1, cnt1, 0, 2)
    accumulate(list2, cnt2, 1, 3)
    pltpu.sync_copy(uacc, out_hbm.at[c, s])


ROW_BLK = 400
N_BLK = N // ROW_BLK


def _tc_dinv(deg_part):
    def body(p_ref, o_ref):
        deg = jnp.sum(p_ref[...], axis=0) + 1.0
        o_ref[...] = lax.rsqrt(deg)

    return pl.pallas_call(
        body,
        grid=(N_BLK,),
        in_specs=[pl.BlockSpec((NW, ROW_BLK, 1), lambda i: (0, i, 0))],
        out_specs=pl.BlockSpec((ROW_BLK, 1), lambda i: (i, 0)),
        out_shape=jax.ShapeDtypeStruct((N, 1), jnp.float32),
    )(deg_part)


def _tc_pre(x, dinv, W):
    def body(x_ref, d_ref, w_ref, o_ref):
        o_ref[...] = jnp.dot(d_ref[...] * x_ref[...], w_ref[...],
                             preferred_element_type=jnp.float32)

    return pl.pallas_call(
        body,
        grid=(N_BLK,),
        in_specs=[
            pl.BlockSpec((ROW_BLK, D), lambda i: (i, 0)),
            pl.BlockSpec((ROW_BLK, 1), lambda i: (i, 0)),
            pl.BlockSpec((D, D), lambda i: (0, 0)),
        ],
        out_specs=pl.BlockSpec((ROW_BLK, D), lambda i: (i, 0)),
        out_shape=jax.ShapeDtypeStruct((N, D), jnp.float32),
    )(x, dinv, W)


def _tc_q(part, z, dinv, b):
    def body(p_ref, z_ref, d_ref, b_ref, o_ref):
        agg = p_ref[0] + p_ref[1] + z_ref[...]
        h = jnp.maximum(d_ref[...] * agg + b_ref[...], 0.0)
        o_ref[...] = d_ref[...] * h

    return pl.pallas_call(
        body,
        grid=(N_BLK,),
        in_specs=[
            pl.BlockSpec((NC, ROW_BLK, D), lambda i: (0, i, 0)),
            pl.BlockSpec((ROW_BLK, D), lambda i: (i, 0)),
            pl.BlockSpec((ROW_BLK, 1), lambda i: (i, 0)),
            pl.BlockSpec((1, D), lambda i: (0, 0)),
        ],
        out_specs=pl.BlockSpec((ROW_BLK, D), lambda i: (i, 0)),
        out_shape=jax.ShapeDtypeStruct((N, D), jnp.float32),
    )(part, z, dinv, b)


def _tc_finish(slabs, q1, q2, dinv, W1b, b1b, W2b, b2b, ops):
    def body(ops_ref, sl_ref, q1_ref, q2_ref, d_ref, w1_ref, b1_ref,
             w2_ref, b2_ref, o_ref):
        u = jnp.sum(sl_ref[...], axis=0)
        o1 = ops_ref[0]
        o2 = ops_ref[1]
        u11 = u[0:1] + q1_ref[pl.ds(o1, 1), :]
        u12 = u[1:2] + q1_ref[pl.ds(o2, 1), :]
        u21 = u[2:3] + q2_ref[pl.ds(o1, 1), :]
        u22 = u[3:4] + q2_ref[pl.ds(o2, 1), :]
        d1 = d_ref[pl.ds(o1, 1), :]
        d2 = d_ref[pl.ds(o2, 1), :]
        a = jnp.dot(jnp.concatenate([u11, u12], axis=0), w1_ref[...],
                    preferred_element_type=jnp.float32)
        b = jnp.dot(jnp.concatenate([u21, u22], axis=0), w2_ref[...],
                    preferred_element_type=jnp.float32)
        y11 = d1 * a[0:1] + b1_ref[...]
        y12 = d2 * a[1:2] + b1_ref[...]
        y21 = d1 * b[0:1] + b2_ref[...]
        y22 = d2 * b[1:2] + b2_ref[...]
        o_ref[0] = jnp.sum(y11 * y12)
        o_ref[1] = jnp.sum(y21 * y22)

    return pl.pallas_call(
        body,
        in_specs=[
            pl.BlockSpec(memory_space=pltpu.SMEM),
            pl.BlockSpec(memory_space=pltpu.VMEM),
            pl.BlockSpec(memory_space=pltpu.VMEM),
            pl.BlockSpec(memory_space=pltpu.VMEM),
            pl.BlockSpec(memory_space=pltpu.VMEM),
            pl.BlockSpec(memory_space=pltpu.VMEM),
            pl.BlockSpec(memory_space=pltpu.VMEM),
            pl.BlockSpec(memory_space=pltpu.VMEM),
            pl.BlockSpec(memory_space=pltpu.VMEM),
        ],
        out_specs=pl.BlockSpec(memory_space=pltpu.SMEM),
        out_shape=jax.ShapeDtypeStruct((2,), jnp.float32),
    )(ops, slabs, q1, q2, dinv, W1b, b1b, W2b, b2b)


def kernel(x, edge_index, op1, op2, W1a, b1a, W1b, b1b, W2a, b2a, W2b, b2b):
    src = edge_index[0]
    dst = edge_index[1]
    zeros_row = jnp.zeros((ROWS_PER_TILE, D), jnp.float32)
    ops = jnp.stack([op1, op2]).astype(jnp.int32)
    ops16 = jnp.broadcast_to(ops[:, None], (2, L))

    deg_part = _sc_degree(dst)
    dinv = _tc_dinv(deg_part.reshape(NW, NP, 1))

    dst_t = dst.reshape(NW, N_CHUNK, CH)
    z1 = _tc_pre(x, dinv, W1a)
    z2 = _tc_pre(x, dinv, W2a)
    p1 = _sc_edge_pass(z1, src, dst_t, zeros_row)
    p2 = _sc_edge_pass(z2, src, dst_t, zeros_row)
    q1 = _tc_q(p1, z1, dinv, b1a.reshape(1, D))
    q2 = _tc_q(p2, z2, dinv, b2a.reshape(1, D))

    slabs = _sc_filter(src, dst, ops16, q1, q2)
    slabs = slabs.reshape(NC * NS, 8, D)
    return _tc_finish(slabs, q1, q2, dinv, W1b, b1b.reshape(1, D),
                      W2b, b2b.reshape(1, D), ops)


# speedup vs baseline: 24.6643x; 1.0672x over previous
import functools

import jax
import jax.numpy as jnp
from jax import lax
from jax.experimental import pallas as pl
from jax.experimental.pallas import tpu as pltpu
from jax.experimental.pallas import tpu_sc as plsc

N = 10000
E = 320000
D = 128

NC = 2
NS = 16
NW = NC * NS
PER_TILE = E // NW
CH = 80
N_CHUNK = PER_TILE // CH
NP = 10112
ROWS_PER_TILE = NP // NS

_MESH = plsc.VectorSubcoreMesh(core_axis_name="c", subcore_axis_name="s")

L = 16
NVEC = PER_TILE // L
NPVEC = NP // L


@functools.partial(
    pl.kernel,
    out_type=jax.ShapeDtypeStruct((NW, NP), jnp.float32),
    mesh=_MESH,
    compiler_params=pltpu.CompilerParams(needs_layout_passes=False),
    scratch_types=[
        pltpu.VMEM((PER_TILE,), jnp.int32),
        pltpu.VMEM((NP,), jnp.float32),
    ],
)
def _sc_degree(dst_hbm, out_hbm, dst_b, hist):
    c = lax.axis_index("c")
    s = lax.axis_index("s")
    wid = s * NC + c
    pltpu.sync_copy(dst_hbm.at[pl.ds(wid * PER_TILE, PER_TILE)], dst_b)
    z16 = jnp.zeros((L,), jnp.float32)
    one16 = jnp.ones((L,), jnp.float32)

    def zbody(i, carry):
        hist[pl.ds(i * L, L)] = z16
        return carry

    lax.fori_loop(0, NPVEC, zbody, 0)

    def body(i, carry):
        idx = dst_b[pl.ds(i * L, L)]
        plsc.addupdate_scatter(hist, [idx], one16)
        return carry

    lax.fori_loop(0, NVEC, body, 0)
    pltpu.sync_copy(hist, out_hbm.at[wid])


@functools.partial(
    pl.kernel,
    out_type=jax.ShapeDtypeStruct((NC, NP, D), jnp.float32),
    mesh=_MESH,
    scratch_types=[
        pltpu.VMEM((PER_TILE,), jnp.int32),
        pltpu.VMEM((N_CHUNK, CH), jnp.int32),
        pltpu.VMEM((2, CH, D), jnp.float32),
        pltpu.VMEM_SHARED((NP, D), jnp.float32),
        pltpu.SemaphoreType.DMA,
        pltpu.SemaphoreType.DMA,
    ],
)
def _sc_edge_pass(z_hbm, src_hbm, dst_hbm, zeros_hbm, out_hbm,
                  src_v, dst_v, rows_v, acc_s, semA, semB):
    c = lax.axis_index("c")
    s = lax.axis_index("s")
    wid = s * NC + c
    pltpu.sync_copy(zeros_hbm, acc_s.at[pl.ds(s * ROWS_PER_TILE, ROWS_PER_TILE)])
    pltpu.sync_copy(src_hbm.at[pl.ds(wid * PER_TILE, PER_TILE)], src_v)
    pltpu.sync_copy(dst_hbm.at[wid], dst_v)
    plsc.subcore_barrier()

    def gather_start(i, p, sem):
        pltpu.async_copy(z_hbm.at[src_v.at[pl.ds(i * CH, CH)]], rows_v.at[p], sem)

    def gather_wait(i, p, sem):
        pltpu.make_async_copy(
            z_hbm.at[src_v.at[pl.ds(i * CH, CH)]], rows_v.at[p], sem).wait()

    def scatter(i, p):
        pltpu.sync_copy(rows_v.at[p], acc_s.at[dst_v.at[i]], add=True)

    gather_start(0, 0, semA)

    def body(i, carry):
        @pl.when(i % 2 == 0)
        def _():
            gather_start(i + 1, 1, semB)
            gather_wait(i, 0, semA)
            scatter(i, 0)

        @pl.when(i % 2 == 1)
        def _():
            gather_start(i + 1, 0, semA)
            gather_wait(i, 1, semB)
            scatter(i, 1)

        return carry

    lax.fori_loop(0, N_CHUNK - 1, body, 0)
    gather_wait(N_CHUNK - 1, 0, semA)
    scatter(N_CHUNK - 1, 0)
    plsc.subcore_barrier()
    pltpu.sync_copy(
        acc_s.at[pl.ds(s * ROWS_PER_TILE, ROWS_PER_TILE)],
        out_hbm.at[c, pl.ds(s * ROWS_PER_TILE, ROWS_PER_TILE)],
    )


LCAP = PER_TILE + 2 * L
TRASH = LCAP - 1


@functools.partial(
    pl.kernel,
    out_type=jax.ShapeDtypeStruct((NC, NS, 8, D), jnp.float32),
    mesh=_MESH,
    compiler_params=pltpu.CompilerParams(needs_layout_passes=False),
    scratch_types=[
        pltpu.VMEM((PER_TILE,), jnp.int32),
        pltpu.VMEM((PER_TILE,), jnp.int32),
        pltpu.VMEM((LCAP,), jnp.int32),
        pltpu.VMEM((LCAP,), jnp.int32),
        pltpu.VMEM((2, L), jnp.int32),
        pltpu.VMEM((L,), jnp.int32),
        pltpu.VMEM((2,), jnp.int32),
        pltpu.VMEM((L, D), jnp.float32),
        pltpu.VMEM((L, D), jnp.float32),
        pltpu.VMEM((8, D), jnp.float32),
        pltpu.SemaphoreType.DMA,
    ],
)
def _sc_filter(src_hbm, dst_hbm, ops_hbm, ops2_hbm, q1_hbm, q2_hbm, out_hbm,
               dst_b, src_b, list1, list2, ops_v, pad_v, idx2_v, rows1, rows2,
               uacc, sem):
    c = lax.axis_index("c")
    s = lax.axis_index("s")
    wid = s * NC + c
    base = wid * PER_TILE
    pltpu.sync_copy(dst_hbm.at[pl.ds(base, PER_TILE)], dst_b)
    pltpu.sync_copy(src_hbm.at[pl.ds(base, PER_TILE)], src_b)
    pltpu.sync_copy(ops_hbm, ops_v)
    op1v = ops_v[0, :]
    op2v = ops_v[1, :]
    z16 = jnp.zeros((L,), jnp.float32)
    for r in range(8):
        for j in range(D // L):
            uacc[r, pl.ds(j * L, L)] = z16

    def process_vreg(i, c1, c2):
        dv = dst_b[pl.ds(i * L, L)]
        sv = src_b[pl.ds(i * L, L)]
        m1 = dv == op1v
        m2 = dv == op2v
        one16 = jnp.ones((L,), jnp.int32)
        trash16 = jnp.full((L,), TRASH, jnp.int32)
        pos1 = plsc.cumsum(m1.astype(jnp.int32))
        pos2 = plsc.cumsum(m2.astype(jnp.int32))
        c1v = jnp.full((L,), c1, jnp.int32)
        c2v = jnp.full((L,), c2, jnp.int32)
        idx1 = jnp.where(m1, c1v + pos1 - one16, trash16)
        idx2 = jnp.where(m2, c2v + pos2 - one16, trash16)
        plsc.store_scatter(list1, [idx1], sv)
        plsc.store_scatter(list2, [idx2], sv)
        return c1 + jnp.max(pos1), c2 + jnp.max(pos2)

    VPC = 25

    def scan_chunk(ci, carry):
        anym = jnp.zeros((L,), jnp.int32) > jnp.zeros((L,), jnp.int32)
        for j in range(VPC):
            dv = dst_b[pl.ds((ci * VPC + j) * L, L)]
            anym = anym | (dv == op1v) | (dv == op2v)
        pc = plsc.all_reduce_population_count(anym)

        def slow(c1, c2):
            for j in range(VPC):
                c1, c2 = process_vreg(ci * VPC + j, c1, c2)
            return c1, c2

        def fast(c1, c2):
            return c1, c2

        return lax.cond(pc[0] > 0, slow, fast, *carry)

    cnt1, cnt2 = lax.fori_loop(0, NVEC // VPC, scan_chunk,
                               (jnp.int32(0), jnp.int32(0)))

    zi16 = jnp.zeros((L,), jnp.int32)
    pad_v[...] = zi16

    def accumulate(lst, cnt, r1, r2):
        lst[pl.ds(cnt, L)] = zi16
        nch = (cnt + (L - 1)) // L

        def body(k, carry):
            ids = lst.at[pl.ds(k * L, L)]
            pltpu.async_copy(q1_hbm.at[ids], rows1, sem).wait()
            pltpu.async_copy(q2_hbm.at[ids], rows2, sem).wait()
            for j in range(D // L):
                sl = pl.ds(j * L, L)
                a1 = uacc[r1, sl]
                a2 = uacc[r2, sl]
                for t in range(L):
                    a1 = a1 + rows1[t, sl]
                    a2 = a2 + rows2[t, sl]
                uacc[r1, sl] = a1
                uacc[r2, sl] = a2
            return carry

        lax.fori_loop(0, nch, body, 0)
        nspv = jnp.full((L,), (nch * L - cnt).astype(jnp.float32))
        pltpu.async_copy(q1_hbm.at[pad_v], rows1, sem).wait()
        pltpu.async_copy(q2_hbm.at[pad_v], rows2, sem).wait()
        for j in range(D // L):
            sl = pl.ds(j * L, L)
            uacc[r1, sl] = uacc[r1, sl] - nspv * rows1[0, sl]
            uacc[r2, sl] = uacc[r2, sl] - nspv * rows2[0, sl]

    accumulate(list1, cnt1, 0, 2)
    accumulate(list2, cnt2, 1, 3)

    @pl.when(wid == 0)
    def _():
        pltpu.sync_copy(ops2_hbm, idx2_v)
        pltpu.async_copy(q1_hbm.at[idx2_v], rows1.at[pl.ds(0, 2)], sem).wait()
        pltpu.async_copy(q2_hbm.at[idx2_v], rows2.at[pl.ds(0, 2)], sem).wait()
        for j in range(D // L):
            sl = pl.ds(j * L, L)
            uacc[4, sl] = rows1[0, sl]
            uacc[5, sl] = rows1[1, sl]
            uacc[6, sl] = rows2[0, sl]
            uacc[7, sl] = rows2[1, sl]

    pltpu.sync_copy(uacc, out_hbm.at[c, s])


ROW_BLK = 400
N_BLK = N // ROW_BLK


def _tc_pre(x, deg_part, W1, W2):
    def body(x_ref, p_ref, w1_ref, w2_ref, o1_ref, o2_ref, d_ref):
        deg = jnp.sum(p_ref[...], axis=0) + 1.0
        d = lax.rsqrt(deg)
        d_ref[...] = d
        xd = d * x_ref[...]
        o1_ref[...] = jnp.dot(xd, w1_ref[...],
                              preferred_element_type=jnp.float32)
        o2_ref[...] = jnp.dot(xd, w2_ref[...],
                              preferred_element_type=jnp.float32)

    return pl.pallas_call(
        body,
        grid=(N_BLK,),
        in_specs=[
            pl.BlockSpec((ROW_BLK, D), lambda i: (i, 0)),
            pl.BlockSpec((NW, ROW_BLK, 1), lambda i: (0, i, 0)),
            pl.BlockSpec((D, D), lambda i: (0, 0)),
            pl.BlockSpec((D, D), lambda i: (0, 0)),
        ],
        out_specs=[
            pl.BlockSpec((ROW_BLK, D), lambda i: (i, 0)),
            pl.BlockSpec((ROW_BLK, D), lambda i: (i, 0)),
            pl.BlockSpec((ROW_BLK, 1), lambda i: (i, 0)),
        ],
        out_shape=[
            jax.ShapeDtypeStruct((N, D), jnp.float32),
            jax.ShapeDtypeStruct((N, D), jnp.float32),
            jax.ShapeDtypeStruct((N, 1), jnp.float32),
        ],
    )(x, deg_part, W1, W2)


def _tc_q(p1, p2, z1, z2, dinv, b1, b2):
    def body(p1_ref, p2_ref, z1_ref, z2_ref, d_ref, b1_ref, b2_ref,
             o1_ref, o2_ref):
        d = d_ref[...]
        h1 = jnp.maximum(d * (p1_ref[0] + p1_ref[1] + z1_ref[...])
                         + b1_ref[...], 0.0)
        h2 = jnp.maximum(d * (p2_ref[0] + p2_ref[1] + z2_ref[...])
                         + b2_ref[...], 0.0)
        o1_ref[...] = d * h1
        o2_ref[...] = d * h2

    return pl.pallas_call(
        body,
        grid=(N_BLK,),
        in_specs=[
            pl.BlockSpec((NC, ROW_BLK, D), lambda i: (0, i, 0)),
            pl.BlockSpec((NC, ROW_BLK, D), lambda i: (0, i, 0)),
            pl.BlockSpec((ROW_BLK, D), lambda i: (i, 0)),
            pl.BlockSpec((ROW_BLK, D), lambda i: (i, 0)),
            pl.BlockSpec((ROW_BLK, 1), lambda i: (i, 0)),
            pl.BlockSpec((1, D), lambda i: (0, 0)),
            pl.BlockSpec((1, D), lambda i: (0, 0)),
        ],
        out_specs=[
            pl.BlockSpec((ROW_BLK, D), lambda i: (i, 0)),
            pl.BlockSpec((ROW_BLK, D), lambda i: (i, 0)),
        ],
        out_shape=[
            jax.ShapeDtypeStruct((N, D), jnp.float32),
            jax.ShapeDtypeStruct((N, D), jnp.float32),
        ],
    )(p1, p2, z1, z2, dinv, b1, b2)


def _tc_finish(slabs, dinv, W1b, b1b, W2b, b2b, ops):
    def body(ops_ref, sl_ref, d_ref, w1_ref, b1_ref, w2_ref, b2_ref, o_ref):
        u = jnp.sum(sl_ref[...], axis=0)
        o1 = ops_ref[0]
        o2 = ops_ref[1]
        u1 = u[0:2] + u[4:6]
        u2 = u[2:4] + u[6:8]
        d1 = d_ref[pl.ds(o1, 1), :]
        d2 = d_ref[pl.ds(o2, 1), :]
        a = jnp.dot(u1, w1_ref[...], preferred_element_type=jnp.float32)
        b = jnp.dot(u2, w2_ref[...], preferred_element_type=jnp.float32)
        y11 = d1 * a[0:1] + b1_ref[...]
        y12 = d2 * a[1:2] + b1_ref[...]
        y21 = d1 * b[0:1] + b2_ref[...]
        y22 = d2 * b[1:2] + b2_ref[...]
        o_ref[0] = jnp.sum(y11 * y12)
        o_ref[1] = jnp.sum(y21 * y22)

    return pl.pallas_call(
        body,
        in_specs=[
            pl.BlockSpec(memory_space=pltpu.SMEM),
            pl.BlockSpec(memory_space=pltpu.VMEM),
            pl.BlockSpec(memory_space=pltpu.VMEM),
            pl.BlockSpec(memory_space=pltpu.VMEM),
            pl.BlockSpec(memory_space=pltpu.VMEM),
            pl.BlockSpec(memory_space=pltpu.VMEM),
            pl.BlockSpec(memory_space=pltpu.VMEM),
        ],
        out_specs=pl.BlockSpec(memory_space=pltpu.SMEM),
        out_shape=jax.ShapeDtypeStruct((2,), jnp.float32),
    )(ops, slabs, dinv, W1b, b1b, W2b, b2b)


def kernel(x, edge_index, op1, op2, W1a, b1a, W1b, b1b, W2a, b2a, W2b, b2b):
    src = edge_index[0]
    dst = edge_index[1]
    zeros_row = jnp.zeros((ROWS_PER_TILE, D), jnp.float32)
    ops = jnp.stack([op1, op2]).astype(jnp.int32)
    ops16 = jnp.broadcast_to(ops[:, None], (2, L))

    deg_part = _sc_degree(dst)
    dst_t = dst.reshape(NW, N_CHUNK, CH)
    z1, z2, dinv = _tc_pre(x, deg_part.reshape(NW, NP, 1), W1a, W2a)
    p1 = _sc_edge_pass(z1, src, dst_t, zeros_row)
    p2 = _sc_edge_pass(z2, src, dst_t, zeros_row)
    q1, q2 = _tc_q(p1, p2, z1, z2, dinv, b1a.reshape(1, D), b2a.reshape(1, D))

    slabs = _sc_filter(src, dst, ops16, ops, q1, q2)
    slabs = slabs.reshape(NC * NS, 8, D)
    return _tc_finish(slabs, dinv, W1b, b1b.reshape(1, D),
                      W2b, b2b.reshape(1, D), ops)


# speedup vs baseline: 25.1532x; 1.0198x over previous
import functools

import jax
import jax.numpy as jnp
from jax import lax
from jax.experimental import pallas as pl
from jax.experimental.pallas import tpu as pltpu
from jax.experimental.pallas import tpu_sc as plsc

N = 10000
E = 320000
D = 128

NC = 2
NS = 16
NW = NC * NS
PER_TILE = E // NW
CH = 80
N_CHUNK = PER_TILE // CH
NP = 10112
ROWS_PER_TILE = NP // NS

_MESH = plsc.VectorSubcoreMesh(core_axis_name="c", subcore_axis_name="s")

L = 16
NVEC = PER_TILE // L
NPVEC = NP // L


@functools.partial(
    pl.kernel,
    out_type=jax.ShapeDtypeStruct((NW, NP), jnp.float32),
    mesh=_MESH,
    compiler_params=pltpu.CompilerParams(needs_layout_passes=False),
    scratch_types=[
        pltpu.VMEM((PER_TILE,), jnp.int32),
        pltpu.VMEM((NP,), jnp.float32),
    ],
)
def _sc_degree(dst_hbm, out_hbm, dst_b, hist):
    c = lax.axis_index("c")
    s = lax.axis_index("s")
    wid = s * NC + c
    pltpu.sync_copy(dst_hbm.at[pl.ds(wid * PER_TILE, PER_TILE)], dst_b)
    z16 = jnp.zeros((L,), jnp.float32)
    one16 = jnp.ones((L,), jnp.float32)

    def zbody(i, carry):
        hist[pl.ds(i * L, L)] = z16
        return carry

    lax.fori_loop(0, NPVEC, zbody, 0)

    def body(i, carry):
        idx = dst_b[pl.ds(i * L, L)]
        plsc.addupdate_scatter(hist, [idx], one16)
        return carry

    lax.fori_loop(0, NVEC, body, 0)
    pltpu.sync_copy(hist, out_hbm.at[wid])


@functools.partial(
    pl.kernel,
    out_type=[
        jax.ShapeDtypeStruct((NC, NP, D), jnp.float32),
        jax.ShapeDtypeStruct((NC, NP, D), jnp.float32),
    ],
    mesh=_MESH,
    scratch_types=[
        pltpu.VMEM((PER_TILE,), jnp.int32),
        pltpu.VMEM((N_CHUNK, CH), jnp.int32),
        pltpu.VMEM((2, CH, D), jnp.float32),
        pltpu.VMEM_SHARED((NP, D), jnp.float32),
        pltpu.SemaphoreType.DMA,
        pltpu.SemaphoreType.DMA,
    ],
)
def _sc_edge_pass(z1_hbm, z2_hbm, src_hbm, dst_hbm, zeros_hbm,
                  out1_hbm, out2_hbm, src_v, dst_v, rows_v, acc_s, semA, semB):
    c = lax.axis_index("c")
    s = lax.axis_index("s")
    wid = s * NC + c
    rows = pl.ds(s * ROWS_PER_TILE, ROWS_PER_TILE)
    pltpu.sync_copy(zeros_hbm, acc_s.at[rows])
    pltpu.sync_copy(src_hbm.at[pl.ds(wid * PER_TILE, PER_TILE)], src_v)
    pltpu.sync_copy(dst_hbm.at[wid], dst_v)
    plsc.subcore_barrier()

    def one_pass(z_hbm, out_hbm):
        def gather_start(i, p, sem):
            pltpu.async_copy(z_hbm.at[src_v.at[pl.ds(i * CH, CH)]],
                             rows_v.at[p], sem)

        def gather_wait(i, p, sem):
            pltpu.make_async_copy(
                z_hbm.at[src_v.at[pl.ds(i * CH, CH)]], rows_v.at[p], sem).wait()

        def scatter(i, p):
            pltpu.sync_copy(rows_v.at[p], acc_s.at[dst_v.at[i]], add=True)

        gather_start(0, 0, semA)

        def body(i, carry):
            @pl.when(i % 2 == 0)
            def _():
                gather_start(i + 1, 1, semB)
                gather_wait(i, 0, semA)
                scatter(i, 0)

            @pl.when(i % 2 == 1)
            def _():
                gather_start(i + 1, 0, semA)
                gather_wait(i, 1, semB)
                scatter(i, 1)

            return carry

        lax.fori_loop(0, N_CHUNK - 1, body, 0)
        gather_wait(N_CHUNK - 1, 0, semA)
        scatter(N_CHUNK - 1, 0)
        plsc.subcore_barrier()
        pltpu.sync_copy(acc_s.at[rows], out_hbm.at[c, rows])

    one_pass(z1_hbm, out1_hbm)
    pltpu.sync_copy(zeros_hbm, acc_s.at[rows])
    plsc.subcore_barrier()
    one_pass(z2_hbm, out2_hbm)


LCAP = PER_TILE + 2 * L
TRASH = LCAP - 1


@functools.partial(
    pl.kernel,
    out_type=jax.ShapeDtypeStruct((NC, NS, 8, D), jnp.float32),
    mesh=_MESH,
    compiler_params=pltpu.CompilerParams(needs_layout_passes=False),
    scratch_types=[
        pltpu.VMEM((PER_TILE,), jnp.int32),
        pltpu.VMEM((PER_TILE,), jnp.int32),
        pltpu.VMEM((LCAP,), jnp.int32),
        pltpu.VMEM((LCAP,), jnp.int32),
        pltpu.VMEM((2, L), jnp.int32),
        pltpu.VMEM((L,), jnp.int32),
        pltpu.VMEM((2,), jnp.int32),
        pltpu.VMEM((L, D), jnp.float32),
        pltpu.VMEM((L, D), jnp.float32),
        pltpu.VMEM((8, D), jnp.float32),
        pltpu.SemaphoreType.DMA,
    ],
)
def _sc_filter(src_hbm, dst_hbm, ops_hbm, ops2_hbm, q1_hbm, q2_hbm, out_hbm,
               dst_b, src_b, list1, list2, ops_v, pad_v, idx2_v, rows1, rows2,
               uacc, sem):
    c = lax.axis_index("c")
    s = lax.axis_index("s")
    wid = s * NC + c
    base = wid * PER_TILE
    pltpu.sync_copy(dst_hbm.at[pl.ds(base, PER_TILE)], dst_b)
    pltpu.sync_copy(src_hbm.at[pl.ds(base, PER_TILE)], src_b)
    pltpu.sync_copy(ops_hbm, ops_v)
    op1v = ops_v[0, :]
    op2v = ops_v[1, :]
    z16 = jnp.zeros((L,), jnp.float32)
    for r in range(8):
        for j in range(D // L):
            uacc[r, pl.ds(j * L, L)] = z16

    def process_vreg(i, c1, c2):
        dv = dst_b[pl.ds(i * L, L)]
        sv = src_b[pl.ds(i * L, L)]
        m1 = dv == op1v
        m2 = dv == op2v
        one16 = jnp.ones((L,), jnp.int32)
        trash16 = jnp.full((L,), TRASH, jnp.int32)
        pos1 = plsc.cumsum(m1.astype(jnp.int32))
        pos2 = plsc.cumsum(m2.astype(jnp.int32))
        c1v = jnp.full((L,), c1, jnp.int32)
        c2v = jnp.full((L,), c2, jnp.int32)
        idx1 = jnp.where(m1, c1v + pos1 - one16, trash16)
        idx2 = jnp.where(m2, c2v + pos2 - one16, trash16)
        plsc.store_scatter(list1, [idx1], sv)
        plsc.store_scatter(list2, [idx2], sv)
        return c1 + jnp.max(pos1), c2 + jnp.max(pos2)

    VPC = 25

    def scan_chunk(ci, carry):
        anym = jnp.zeros((L,), jnp.int32) > jnp.zeros((L,), jnp.int32)
        for j in range(VPC):
            dv = dst_b[pl.ds((ci * VPC + j) * L, L)]
            anym = anym | (dv == op1v) | (dv == op2v)
        pc = plsc.all_reduce_population_count(anym)

        def slow(c1, c2):
            return lax.fori_loop(
                ci * VPC, (ci + 1) * VPC,
                lambda j, cc: process_vreg(j, cc[0], cc[1]), (c1, c2))

        def fast(c1, c2):
            return c1, c2

        return lax.cond(pc[0] > 0, slow, fast, *carry)

    cnt1, cnt2 = lax.fori_loop(0, NVEC // VPC, scan_chunk,
                               (jnp.int32(0), jnp.int32(0)))

    zi16 = jnp.zeros((L,), jnp.int32)
    pad_v[...] = zi16

    def accumulate(lst, cnt, r1, r2):
        lst[pl.ds(cnt, L)] = zi16
        nch = (cnt + (L - 1)) // L

        def body(k, carry):
            ids = lst.at[pl.ds(k * L, L)]
            pltpu.async_copy(q1_hbm.at[ids], rows1, sem).wait()
            pltpu.async_copy(q2_hbm.at[ids], rows2, sem).wait()

            def row_add(t, cc):
                for j in range(D // L):
                    sl = pl.ds(j * L, L)
                    uacc[r1, sl] = uacc[r1, sl] + rows1[t, sl]
                    uacc[r2, sl] = uacc[r2, sl] + rows2[t, sl]
                return cc

            lax.fori_loop(0, L, row_add, 0)
            return carry

        lax.fori_loop(0, nch, body, 0)
        nspv = jnp.full((L,), (nch * L - cnt).astype(jnp.float32))
        pltpu.async_copy(q1_hbm.at[pad_v], rows1, sem).wait()
        pltpu.async_copy(q2_hbm.at[pad_v], rows2, sem).wait()
        for j in range(D // L):
            sl = pl.ds(j * L, L)
            uacc[r1, sl] = uacc[r1, sl] - nspv * rows1[0, sl]
            uacc[r2, sl] = uacc[r2, sl] - nspv * rows2[0, sl]

    accumulate(list1, cnt1, 0, 2)
    accumulate(list2, cnt2, 1, 3)

    @pl.when(wid == 0)
    def _():
        pltpu.sync_copy(ops2_hbm, idx2_v)
        pltpu.async_copy(q1_hbm.at[idx2_v], rows1.at[pl.ds(0, 2)], sem).wait()
        pltpu.async_copy(q2_hbm.at[idx2_v], rows2.at[pl.ds(0, 2)], sem).wait()
        for j in range(D // L):
            sl = pl.ds(j * L, L)
            uacc[4, sl] = rows1[0, sl]
            uacc[5, sl] = rows1[1, sl]
            uacc[6, sl] = rows2[0, sl]
            uacc[7, sl] = rows2[1, sl]

    pltpu.sync_copy(uacc, out_hbm.at[c, s])


ROW_BLK = 400
N_BLK = N // ROW_BLK


def _tc_pre(x, deg_part, W1, W2):
    def body(x_ref, p_ref, w1_ref, w2_ref, o1_ref, o2_ref, d_ref):
        deg = jnp.sum(p_ref[...], axis=0) + 1.0
        d = lax.rsqrt(deg)
        d_ref[...] = d
        xd = d * x_ref[...]
        o1_ref[...] = jnp.dot(xd, w1_ref[...],
                              preferred_element_type=jnp.float32)
        o2_ref[...] = jnp.dot(xd, w2_ref[...],
                              preferred_element_type=jnp.float32)

    return pl.pallas_call(
        body,
        grid=(N_BLK,),
        in_specs=[
            pl.BlockSpec((ROW_BLK, D), lambda i: (i, 0)),
            pl.BlockSpec((NW, ROW_BLK, 1), lambda i: (0, i, 0)),
            pl.BlockSpec((D, D), lambda i: (0, 0)),
            pl.BlockSpec((D, D), lambda i: (0, 0)),
        ],
        out_specs=[
            pl.BlockSpec((ROW_BLK, D), lambda i: (i, 0)),
            pl.BlockSpec((ROW_BLK, D), lambda i: (i, 0)),
            pl.BlockSpec((ROW_BLK, 1), lambda i: (i, 0)),
        ],
        out_shape=[
            jax.ShapeDtypeStruct((N, D), jnp.float32),
            jax.ShapeDtypeStruct((N, D), jnp.float32),
            jax.ShapeDtypeStruct((N, 1), jnp.float32),
        ],
    )(x, deg_part, W1, W2)


def _tc_q(p1, p2, z1, z2, dinv, b1, b2):
    def body(p1_ref, p2_ref, z1_ref, z2_ref, d_ref, b1_ref, b2_ref,
             o1_ref, o2_ref):
        d = d_ref[...]
        h1 = jnp.maximum(d * (p1_ref[0] + p1_ref[1] + z1_ref[...])
                         + b1_ref[...], 0.0)
        h2 = jnp.maximum(d * (p2_ref[0] + p2_ref[1] + z2_ref[...])
                         + b2_ref[...], 0.0)
        o1_ref[...] = d * h1
        o2_ref[...] = d * h2

    return pl.pallas_call(
        body,
        grid=(N_BLK,),
        in_specs=[
            pl.BlockSpec((NC, ROW_BLK, D), lambda i: (0, i, 0)),
            pl.BlockSpec((NC, ROW_BLK, D), lambda i: (0, i, 0)),
            pl.BlockSpec((ROW_BLK, D), lambda i: (i, 0)),
            pl.BlockSpec((ROW_BLK, D), lambda i: (i, 0)),
            pl.BlockSpec((ROW_BLK, 1), lambda i: (i, 0)),
            pl.BlockSpec((1, D), lambda i: (0, 0)),
            pl.BlockSpec((1, D), lambda i: (0, 0)),
        ],
        out_specs=[
            pl.BlockSpec((ROW_BLK, D), lambda i: (i, 0)),
            pl.BlockSpec((ROW_BLK, D), lambda i: (i, 0)),
        ],
        out_shape=[
            jax.ShapeDtypeStruct((N, D), jnp.float32),
            jax.ShapeDtypeStruct((N, D), jnp.float32),
        ],
    )(p1, p2, z1, z2, dinv, b1, b2)


def _tc_finish(slabs, dinv, W1b, b1b, W2b, b2b, ops):
    def body(ops_ref, sl_ref, d_ref, w1_ref, b1_ref, w2_ref, b2_ref, o_ref):
        u = jnp.sum(sl_ref[...], axis=0)
        o1 = ops_ref[0]
        o2 = ops_ref[1]
        u1 = u[0:2] + u[4:6]
        u2 = u[2:4] + u[6:8]
        d1 = d_ref[pl.ds(o1, 1), :]
        d2 = d_ref[pl.ds(o2, 1), :]
        a = jnp.dot(u1, w1_ref[...], preferred_element_type=jnp.float32)
        b = jnp.dot(u2, w2_ref[...], preferred_element_type=jnp.float32)
        y11 = d1 * a[0:1] + b1_ref[...]
        y12 = d2 * a[1:2] + b1_ref[...]
        y21 = d1 * b[0:1] + b2_ref[...]
        y22 = d2 * b[1:2] + b2_ref[...]
        o_ref[0] = jnp.sum(y11 * y12)
        o_ref[1] = jnp.sum(y21 * y22)

    return pl.pallas_call(
        body,
        in_specs=[
            pl.BlockSpec(memory_space=pltpu.SMEM),
            pl.BlockSpec(memory_space=pltpu.VMEM),
            pl.BlockSpec(memory_space=pltpu.VMEM),
            pl.BlockSpec(memory_space=pltpu.VMEM),
            pl.BlockSpec(memory_space=pltpu.VMEM),
            pl.BlockSpec(memory_space=pltpu.VMEM),
            pl.BlockSpec(memory_space=pltpu.VMEM),
        ],
        out_specs=pl.BlockSpec(memory_space=pltpu.SMEM),
        out_shape=jax.ShapeDtypeStruct((2,), jnp.float32),
    )(ops, slabs, dinv, W1b, b1b, W2b, b2b)


def kernel(x, edge_index, op1, op2, W1a, b1a, W1b, b1b, W2a, b2a, W2b, b2b):
    src = edge_index[0]
    dst = edge_index[1]
    zeros_row = jnp.zeros((ROWS_PER_TILE, D), jnp.float32)
    ops = jnp.stack([op1, op2]).astype(jnp.int32)
    ops16 = jnp.broadcast_to(ops[:, None], (2, L))

    deg_part = _sc_degree(dst)
    dst_t = dst.reshape(NW, N_CHUNK, CH)
    z1, z2, dinv = _tc_pre(x, deg_part.reshape(NW, NP, 1), W1a, W2a)
    p1, p2 = _sc_edge_pass(z1, z2, src, dst_t, zeros_row)
    q1, q2 = _tc_q(p1, p2, z1, z2, dinv, b1a.reshape(1, D), b2a.reshape(1, D))

    slabs = _sc_filter(src, dst, ops16, ops, q1, q2)
    slabs = slabs.reshape(NC * NS, 8, D)
    return _tc_finish(slabs, dinv, W1b, b1b.reshape(1, D),
                      W2b, b2b.reshape(1, D), ops)


# speedup vs baseline: 27.1306x; 1.0786x over previous
import functools

import jax
import jax.numpy as jnp
from jax import lax
from jax.experimental import pallas as pl
from jax.experimental.pallas import tpu as pltpu
from jax.experimental.pallas import tpu_sc as plsc

N = 10000
E = 320000
D = 128

NC = 2
NS = 16
NW = NC * NS
PER_TILE = E // NW
CH = 80
N_CHUNK = PER_TILE // CH
NP = 10112
ROWS_PER_TILE = NP // NS

_MESH = plsc.VectorSubcoreMesh(core_axis_name="c", subcore_axis_name="s")

L = 16
NVEC = PER_TILE // L
NPVEC = NP // L


@functools.partial(
    pl.kernel,
    out_type=jax.ShapeDtypeStruct((NW, NP), jnp.float32),
    mesh=_MESH,
    compiler_params=pltpu.CompilerParams(needs_layout_passes=False),
    scratch_types=[
        pltpu.VMEM((PER_TILE,), jnp.int32),
        pltpu.VMEM((NP,), jnp.float32),
    ],
)
def _sc_degree(dst_hbm, out_hbm, dst_b, hist):
    c = lax.axis_index("c")
    s = lax.axis_index("s")
    wid = s * NC + c
    pltpu.sync_copy(dst_hbm.at[pl.ds(wid * PER_TILE, PER_TILE)], dst_b)
    z16 = jnp.zeros((L,), jnp.float32)
    one16 = jnp.ones((L,), jnp.float32)

    def zbody(i, carry):
        hist[pl.ds(i * L, L)] = z16
        return carry

    lax.fori_loop(0, NPVEC, zbody, 0)

    def body(i, carry):
        idx = dst_b[pl.ds(i * L, L)]
        plsc.addupdate_scatter(hist, [idx], one16)
        return carry

    lax.fori_loop(0, NVEC, body, 0)
    pltpu.sync_copy(hist, out_hbm.at[wid])


@functools.partial(
    pl.kernel,
    out_type=[
        jax.ShapeDtypeStruct((NC, NP, D), jnp.float32),
        jax.ShapeDtypeStruct((NC, NP, D), jnp.float32),
    ],
    mesh=_MESH,
    scratch_types=[
        pltpu.VMEM((PER_TILE,), jnp.int32),
        pltpu.VMEM((N_CHUNK, CH), jnp.int32),
        pltpu.VMEM((2, CH, D), jnp.float32),
        pltpu.VMEM_SHARED((NP, D), jnp.float32),
        pltpu.SemaphoreType.DMA,
        pltpu.SemaphoreType.DMA,
    ],
)
def _sc_edge_pass(z1_hbm, z2_hbm, src_hbm, dst_hbm, zeros_hbm,
                  out1_hbm, out2_hbm, src_v, dst_v, rows_v, acc_s, semA, semB):
    c = lax.axis_index("c")
    s = lax.axis_index("s")
    wid = s * NC + c
    rows = pl.ds(s * ROWS_PER_TILE, ROWS_PER_TILE)
    pltpu.sync_copy(zeros_hbm, acc_s.at[rows])
    pltpu.sync_copy(src_hbm.at[pl.ds(wid * PER_TILE, PER_TILE)], src_v)
    pltpu.sync_copy(dst_hbm.at[wid], dst_v)
    plsc.subcore_barrier()

    def one_pass(z_hbm, out_hbm):
        def gather_start(i, p, sem):
            pltpu.async_copy(z_hbm.at[src_v.at[pl.ds(i * CH, CH)]],
                             rows_v.at[p], sem)

        def gather_wait(i, p, sem):
            pltpu.make_async_copy(
                z_hbm.at[src_v.at[pl.ds(i * CH, CH)]], rows_v.at[p], sem).wait()

        def scatter(i, p):
            pltpu.sync_copy(rows_v.at[p], acc_s.at[dst_v.at[i]], add=True)

        gather_start(0, 0, semA)

        def body(i, carry):
            @pl.when(i % 2 == 0)
            def _():
                gather_start(i + 1, 1, semB)
                gather_wait(i, 0, semA)
                scatter(i, 0)

            @pl.when(i % 2 == 1)
            def _():
                gather_start(i + 1, 0, semA)
                gather_wait(i, 1, semB)
                scatter(i, 1)

            return carry

        lax.fori_loop(0, N_CHUNK - 1, body, 0)
        gather_wait(N_CHUNK - 1, 0, semA)
        scatter(N_CHUNK - 1, 0)
        plsc.subcore_barrier()
        pltpu.sync_copy(acc_s.at[rows], out_hbm.at[c, rows])

    one_pass(z1_hbm, out1_hbm)
    pltpu.sync_copy(zeros_hbm, acc_s.at[rows])
    plsc.subcore_barrier()
    one_pass(z2_hbm, out2_hbm)


LCAP = PER_TILE + 2 * L
TRASH = LCAP - 1


@functools.partial(
    pl.kernel,
    out_type=jax.ShapeDtypeStruct((NC, NS, 8, D), jnp.float32),
    mesh=_MESH,
    compiler_params=pltpu.CompilerParams(needs_layout_passes=False),
    scratch_types=[
        pltpu.VMEM((PER_TILE,), jnp.int32),
        pltpu.VMEM((PER_TILE,), jnp.int32),
        pltpu.VMEM((LCAP,), jnp.int32),
        pltpu.VMEM((LCAP,), jnp.int32),
        pltpu.VMEM((2, L), jnp.int32),
        pltpu.VMEM((2,), jnp.int32),
        pltpu.VMEM((L, D), jnp.float32),
        pltpu.VMEM((L, D), jnp.float32),
        pltpu.VMEM((8, D), jnp.float32),
        pltpu.SemaphoreType.DMA,
    ],
)
def _sc_filter(src_hbm, dst_hbm, ops_hbm, ops2_hbm, q1_hbm, q2_hbm, out_hbm,
               dst_b, src_b, list1, list2, ops_v, idx2_v, rows1, rows2,
               uacc, sem):
    c = lax.axis_index("c")
    s = lax.axis_index("s")
    wid = s * NC + c
    base = wid * PER_TILE
    pltpu.sync_copy(dst_hbm.at[pl.ds(base, PER_TILE)], dst_b)
    pltpu.sync_copy(src_hbm.at[pl.ds(base, PER_TILE)], src_b)
    pltpu.sync_copy(ops_hbm, ops_v)
    op1v = ops_v[0, :]
    op2v = ops_v[1, :]
    z16 = jnp.zeros((L,), jnp.float32)
    for r in range(8):
        for j in range(D // L):
            uacc[r, pl.ds(j * L, L)] = z16

    def process_vreg(i, c1, c2):
        dv = dst_b[pl.ds(i * L, L)]
        sv = src_b[pl.ds(i * L, L)]
        m1 = dv == op1v
        m2 = dv == op2v
        one16 = jnp.ones((L,), jnp.int32)
        trash16 = jnp.full((L,), TRASH, jnp.int32)
        pos1 = plsc.cumsum(m1.astype(jnp.int32))
        pos2 = plsc.cumsum(m2.astype(jnp.int32))
        c1v = jnp.full((L,), c1, jnp.int32)
        c2v = jnp.full((L,), c2, jnp.int32)
        idx1 = jnp.where(m1, c1v + pos1 - one16, trash16)
        idx2 = jnp.where(m2, c2v + pos2 - one16, trash16)
        plsc.store_scatter(list1, [idx1], sv)
        plsc.store_scatter(list2, [idx2], sv)
        return c1 + jnp.max(pos1), c2 + jnp.max(pos2)

    VPC = 25

    def scan_chunk(ci, carry):
        anym = jnp.zeros((L,), jnp.int32) > jnp.zeros((L,), jnp.int32)
        for j in range(VPC):
            dv = dst_b[pl.ds((ci * VPC + j) * L, L)]
            anym = anym | (dv == op1v) | (dv == op2v)
        pc = plsc.all_reduce_population_count(anym)

        def slow(c1, c2):
            return lax.fori_loop(
                ci * VPC, (ci + 1) * VPC,
                lambda j, cc: process_vreg(j, cc[0], cc[1]), (c1, c2))

        def fast(c1, c2):
            return c1, c2

        return lax.cond(pc[0] > 0, slow, fast, *carry)

    cnt1, cnt2 = lax.fori_loop(0, NVEC // VPC, scan_chunk,
                               (jnp.int32(0), jnp.int32(0)))

    padn16 = jnp.full((L,), N, jnp.int32)

    def accumulate(lst, cnt, r1, r2):
        lst[pl.ds(cnt, L)] = padn16
        nch = (cnt + (L - 1)) // L

        def body(k, carry):
            ids = lst.at[pl.ds(k * L, L)]
            pltpu.async_copy(q1_hbm.at[ids], rows1, sem).wait()
            pltpu.async_copy(q2_hbm.at[ids], rows2, sem).wait()

            def row_add(t, cc):
                for j in range(D // L):
                    sl = pl.ds(j * L, L)
                    uacc[r1, sl] = uacc[r1, sl] + rows1[t, sl]
                    uacc[r2, sl] = uacc[r2, sl] + rows2[t, sl]
                return cc

            lax.fori_loop(0, L, row_add, 0)
            return carry

        lax.fori_loop(0, nch, body, 0)

    accumulate(list1, cnt1, 0, 2)
    accumulate(list2, cnt2, 1, 3)

    @pl.when(wid == 0)
    def _():
        pltpu.sync_copy(ops2_hbm, idx2_v)
        pltpu.async_copy(q1_hbm.at[idx2_v], rows1.at[pl.ds(0, 2)], sem).wait()
        pltpu.async_copy(q2_hbm.at[idx2_v], rows2.at[pl.ds(0, 2)], sem).wait()
        for j in range(D // L):
            sl = pl.ds(j * L, L)
            uacc[4, sl] = rows1[0, sl]
            uacc[5, sl] = rows1[1, sl]
            uacc[6, sl] = rows2[0, sl]
            uacc[7, sl] = rows2[1, sl]

    pltpu.sync_copy(uacc, out_hbm.at[c, s])


ROW_BLK = 400
N_BLK = N // ROW_BLK


def _tc_pre(x, deg_part, W1, W2):
    def body(x_ref, p_ref, w1_ref, w2_ref, o1_ref, o2_ref, d_ref):
        deg = jnp.sum(p_ref[...], axis=0) + 1.0
        d = lax.rsqrt(deg)
        d_ref[...] = d
        xd = d * x_ref[...]
        o1_ref[...] = jnp.dot(xd, w1_ref[...],
                              preferred_element_type=jnp.float32)
        o2_ref[...] = jnp.dot(xd, w2_ref[...],
                              preferred_element_type=jnp.float32)

    return pl.pallas_call(
        body,
        grid=(N_BLK,),
        in_specs=[
            pl.BlockSpec((ROW_BLK, D), lambda i: (i, 0)),
            pl.BlockSpec((NW, ROW_BLK, 1), lambda i: (0, i, 0)),
            pl.BlockSpec((D, D), lambda i: (0, 0)),
            pl.BlockSpec((D, D), lambda i: (0, 0)),
        ],
        out_specs=[
            pl.BlockSpec((ROW_BLK, D), lambda i: (i, 0)),
            pl.BlockSpec((ROW_BLK, D), lambda i: (i, 0)),
            pl.BlockSpec((ROW_BLK, 1), lambda i: (i, 0)),
        ],
        out_shape=[
            jax.ShapeDtypeStruct((N, D), jnp.float32),
            jax.ShapeDtypeStruct((N, D), jnp.float32),
            jax.ShapeDtypeStruct((N, 1), jnp.float32),
        ],
    )(x, deg_part, W1, W2)


NQ = N + ROW_BLK


def _tc_q(p1, p2, z1, z2, dinv, b1, b2):
    def body(p1_ref, p2_ref, z1_ref, z2_ref, d_ref, b1_ref, b2_ref,
             o1_ref, o2_ref):
        k = pl.program_id(0)
        d = d_ref[...]
        h1 = jnp.maximum(d * (p1_ref[0] + p1_ref[1] + z1_ref[...])
                         + b1_ref[...], 0.0)
        h2 = jnp.maximum(d * (p2_ref[0] + p2_ref[1] + z2_ref[...])
                         + b2_ref[...], 0.0)
        live = (k < N_BLK).astype(jnp.float32)
        o1_ref[...] = live * d * h1
        o2_ref[...] = live * d * h2

    clamp = lambda i: (jnp.minimum(i, N_BLK - 1), 0)
    clamp3 = lambda i: (0, jnp.minimum(i, N_BLK - 1), 0)
    return pl.pallas_call(
        body,
        grid=(N_BLK + 1,),
        in_specs=[
            pl.BlockSpec((NC, ROW_BLK, D), clamp3),
            pl.BlockSpec((NC, ROW_BLK, D), clamp3),
            pl.BlockSpec((ROW_BLK, D), clamp),
            pl.BlockSpec((ROW_BLK, D), clamp),
            pl.BlockSpec((ROW_BLK, 1), clamp),
            pl.BlockSpec((1, D), lambda i: (0, 0)),
            pl.BlockSpec((1, D), lambda i: (0, 0)),
        ],
        out_specs=[
            pl.BlockSpec((ROW_BLK, D), lambda i: (i, 0)),
            pl.BlockSpec((ROW_BLK, D), lambda i: (i, 0)),
        ],
        out_shape=[
            jax.ShapeDtypeStruct((NQ, D), jnp.float32),
            jax.ShapeDtypeStruct((NQ, D), jnp.float32),
        ],
    )(p1, p2, z1, z2, dinv, b1, b2)


def _tc_finish(slabs, dinv, W1b, b1b, W2b, b2b, ops):
    def body(ops_ref, sl_ref, d_ref, w1_ref, b1_ref, w2_ref, b2_ref, o_ref):
        u = jnp.sum(sl_ref[...], axis=0)
        o1 = ops_ref[0]
        o2 = ops_ref[1]
        u1 = u[0:2] + u[4:6]
        u2 = u[2:4] + u[6:8]
        d1 = d_ref[pl.ds(o1, 1), :]
        d2 = d_ref[pl.ds(o2, 1), :]
        a = jnp.dot(u1, w1_ref[...], preferred_element_type=jnp.float32)
        b = jnp.dot(u2, w2_ref[...], preferred_element_type=jnp.float32)
        y11 = d1 * a[0:1] + b1_ref[...]
        y12 = d2 * a[1:2] + b1_ref[...]
        y21 = d1 * b[0:1] + b2_ref[...]
        y22 = d2 * b[1:2] + b2_ref[...]
        o_ref[0] = jnp.sum(y11 * y12)
        o_ref[1] = jnp.sum(y21 * y22)

    return pl.pallas_call(
        body,
        in_specs=[
            pl.BlockSpec(memory_space=pltpu.SMEM),
            pl.BlockSpec(memory_space=pltpu.VMEM),
            pl.BlockSpec(memory_space=pltpu.VMEM),
            pl.BlockSpec(memory_space=pltpu.VMEM),
            pl.BlockSpec(memory_space=pltpu.VMEM),
            pl.BlockSpec(memory_space=pltpu.VMEM),
            pl.BlockSpec(memory_space=pltpu.VMEM),
        ],
        out_specs=pl.BlockSpec(memory_space=pltpu.SMEM),
        out_shape=jax.ShapeDtypeStruct((2,), jnp.float32),
    )(ops, slabs, dinv, W1b, b1b, W2b, b2b)


def kernel(x, edge_index, op1, op2, W1a, b1a, W1b, b1b, W2a, b2a, W2b, b2b):
    src = edge_index[0]
    dst = edge_index[1]
    zeros_row = jnp.zeros((ROWS_PER_TILE, D), jnp.float32)
    ops = jnp.stack([op1, op2]).astype(jnp.int32)
    ops16 = jnp.broadcast_to(ops[:, None], (2, L))

    deg_part = _sc_degree(dst)
    dst_t = dst.reshape(NW, N_CHUNK, CH)
    z1, z2, dinv = _tc_pre(x, deg_part.reshape(NW, NP, 1), W1a, W2a)
    p1, p2 = _sc_edge_pass(z1, z2, src, dst_t, zeros_row)
    q1, q2 = _tc_q(p1, p2, z1, z2, dinv, b1a.reshape(1, D), b2a.reshape(1, D))

    slabs = _sc_filter(src, dst, ops16, ops, q1, q2)
    slabs = slabs.reshape(NC * NS, 8, D)
    return _tc_finish(slabs, dinv, W1b, b1b.reshape(1, D),
                      W2b, b2b.reshape(1, D), ops)
